# baseline (speedup 1.0000x reference)
import functools

import jax
import jax.numpy as jnp
from jax import lax
from jax.experimental import pallas as pl
from jax.experimental.pallas import tpu as pltpu
from jax.experimental.pallas import tpu_sc as plsc

N_NODES = 10000
N_R = 16
D = 128
E = 320000

NC = 2
NS = 16
L = 16
NW = NC * NS

PAD_DST = N_NODES
NODES_P = 10016
RPS_A = 632
RPS_T = NODES_P - 15 * RPS_A
CN = 10112 * N_R
CNS = CN // NS
BLK = 128
NBLK = 81
NSTEP = NBLK // 3
EPW = NBLK * BLK
EP = EPW * NW

_mesh = plsc.VectorSubcoreMesh(core_axis_name="c", subcore_axis_name="s")

_GDN = lax.GatherDimensionNumbers(
    offset_dims=(), collapsed_slice_dims=(0,), start_index_map=(0,))


def _dg(v, idx):
    return lax.gather(v, idx[:, None], _GDN, (1,),
                      mode=lax.GatherScatterMode.PROMISE_IN_BOUNDS)


def _splat(v, i):
    return _dg(v, jnp.full((L,), i, jnp.int32))


@functools.partial(
    pl.kernel,
    out_type=jax.ShapeDtypeStruct((NC * CN,), jnp.float32),
    mesh=_mesh,
    scratch_types=[
        pltpu.VMEM((BLK,), jnp.int32),
        pltpu.VMEM((BLK,), jnp.int32),
        pltpu.VMEM((BLK,), jnp.int32),
        pltpu.VMEM((BLK,), jnp.float32),
        pltpu.VMEM_SHARED((CN,), jnp.float32),
    ],
)
def _k_count(dst_hbm, t_hbm, zc_hbm, cnt_hbm, dst_v, t_v, idx_v, ones_v, cnt_sh):
    cid = lax.axis_index("c")
    sid = lax.axis_index("s")
    wid = sid * NC + cid

    for q in range(BLK // L):
        ones_v[pl.ds(q * L, L)] = jnp.full((L,), 1.0, jnp.float32)
    pltpu.sync_copy(zc_hbm.at[pl.ds(sid * CNS, CNS)],
                    cnt_sh.at[pl.ds(sid * CNS, CNS)])
    plsc.subcore_barrier()

    base = wid * EPW

    @pl.loop(0, NBLK)
    def _(blk):
        off = base + blk * BLK
        pltpu.sync_copy(t_hbm.at[pl.ds(off, BLK)], t_v)
        pltpu.sync_copy(dst_hbm.at[pl.ds(off, BLK)], dst_v)
        for q in range(BLK // L):
            sl = pl.ds(q * L, L)
            idx_v[sl] = dst_v[sl] * N_R + t_v[sl]
        pltpu.sync_copy(ones_v, cnt_sh.at[idx_v], add=True)

    plsc.subcore_barrier()
    pltpu.sync_copy(cnt_sh.at[pl.ds(sid * CNS, CNS)],
                    cnt_hbm.at[pl.ds(cid * CN + sid * CNS, CNS)])


@functools.partial(
    pl.kernel,
    out_type=[jax.ShapeDtypeStruct((EP,), jnp.int32),
              jax.ShapeDtypeStruct((EP,), jnp.float32)],
    mesh=_mesh,
    scratch_types=[
        pltpu.VMEM((BLK,), jnp.int32),
        pltpu.VMEM((BLK,), jnp.int32),
        pltpu.VMEM((BLK,), jnp.int32),
        pltpu.VMEM((BLK,), jnp.int32),
        pltpu.VMEM((BLK,), jnp.int32),
        pltpu.VMEM((BLK,), jnp.int32),
        pltpu.VMEM((BLK,), jnp.float32),
        pltpu.VMEM((BLK,), jnp.float32),
        pltpu.VMEM((BLK,), jnp.float32),
        pltpu.SemaphoreType.DMA,
        pltpu.SemaphoreType.DMA,
    ],
)
def _k_scale(src_hbm, dst_hbm, t_hbm, cnt_hbm, g_hbm, sc_hbm,
             src_v, dst_v, t_v, g_v, i0_v, i1_v, c0_v, c1_v, sc_v, sem0, sem1):
    cid = lax.axis_index("c")
    sid = lax.axis_index("s")
    wid = sid * NC + cid
    base = wid * EPW

    @pl.loop(0, NBLK)
    def _(blk):
        off = base + blk * BLK
        pltpu.sync_copy(src_hbm.at[pl.ds(off, BLK)], src_v)
        pltpu.sync_copy(dst_hbm.at[pl.ds(off, BLK)], dst_v)
        pltpu.sync_copy(t_hbm.at[pl.ds(off, BLK)], t_v)
        for q in range(BLK // L):
            sl = pl.ds(q * L, L)
            i0 = dst_v[sl] * N_R + t_v[sl]
            i0_v[sl] = i0
            i1_v[sl] = i0 + CN
            g_v[sl] = t_v[sl] * N_NODES + src_v[sl]
        cp0 = pltpu.async_copy(cnt_hbm.at[i0_v], c0_v, sem0)
        cp1 = pltpu.async_copy(cnt_hbm.at[i1_v], c1_v, sem1)
        cp0.wait()
        cp1.wait()
        for q in range(BLK // L):
            sl = pl.ds(q * L, L)
            sc_v[sl] = 1.0 / jnp.maximum(c0_v[sl] + c1_v[sl], 1.0)
        pltpu.sync_copy(g_v, g_hbm.at[pl.ds(off, BLK)])
        pltpu.sync_copy(sc_v, sc_hbm.at[pl.ds(off, BLK)])


@functools.partial(
    pl.kernel,
    out_type=jax.ShapeDtypeStruct((NC, NODES_P, D), jnp.float32),
    mesh=_mesh,
    scratch_types=(
        [pltpu.VMEM((BLK,), jnp.int32)] * 3
        + [pltpu.VMEM((BLK,), jnp.int32)] * 3
        + [pltpu.VMEM((BLK,), jnp.float32)] * 3
        + [pltpu.VMEM((BLK, D), jnp.float32)] * 3
        + [pltpu.VMEM_SHARED((NODES_P, D), jnp.float32)]
        + [pltpu.SemaphoreType.DMA] * 9
    ),
)
def _k_agg(y_hbm, g_hbm, dst_hbm, sc_hbm, z_hbm, acc_hbm,
           g0, g1, g2, d0, d1, d2, s0, s1, s2, r0, r1, r2, acc_sh,
           si0, si1, si2, sr0, sr1, sr2, ss0, ss1, ss2):
    gv = [g0, g1, g2]
    dv = [d0, d1, d2]
    sv = [s0, s1, s2]
    rv = [r0, r1, r2]
    sem_i = [si0, si1, si2]
    sem_r = [sr0, sr1, sr2]
    sem_s = [ss0, ss1, ss2]

    cid = lax.axis_index("c")
    sid = lax.axis_index("s")
    wid = sid * NC + cid
    base = wid * EPW

    def idx_start(k, p):
        off = base + k * BLK
        pltpu.async_copy(g_hbm.at[pl.ds(off, BLK)], gv[p], sem_i[p])
        pltpu.async_copy(dst_hbm.at[pl.ds(off, BLK)], dv[p], sem_i[p])
        pltpu.async_copy(sc_hbm.at[pl.ds(off, BLK)], sv[p], sem_i[p])

    def idx_wait(p):
        pltpu.make_async_copy(g_hbm.at[pl.ds(0, BLK)], gv[p], sem_i[p]).wait()
        pltpu.make_async_copy(dst_hbm.at[pl.ds(0, BLK)], dv[p], sem_i[p]).wait()
        pltpu.make_async_copy(sc_hbm.at[pl.ds(0, BLK)], sv[p], sem_i[p]).wait()

    def gather_start(p):
        pltpu.async_copy(y_hbm.at[gv[p]], rv[p], sem_r[p])

    def gather_wait(p):
        pltpu.make_async_copy(y_hbm.at[gv[p]], rv[p], sem_r[p]).wait()

    def scat_start(p):
        pltpu.async_copy(rv[p], acc_sh.at[dv[p]], sem_s[p], add=True)

    def scat_wait(p):
        pltpu.make_async_copy(rv[p], acc_sh.at[dv[p]], sem_s[p]).wait()

    @pl.when(sid < 15)
    def _():
        pltpu.sync_copy(z_hbm.at[pl.ds(sid * RPS_A, RPS_A)],
                        acc_sh.at[pl.ds(sid * RPS_A, RPS_A)])

    @pl.when(sid == 15)
    def _():
        pltpu.sync_copy(z_hbm.at[pl.ds(15 * RPS_A, RPS_T)],
                        acc_sh.at[pl.ds(15 * RPS_A, RPS_T)])

    plsc.subcore_barrier()

    idx_start(0, 0)
    idx_start(1, 1)
    idx_wait(0)
    gather_start(0)

    @pl.loop(0, NSTEP)
    def _(step):
        for u in range(3):
            k = step * 3 + u
            p, pn, pn2 = u, (u + 1) % 3, (u + 2) % 3

            if u == 0:
                @pl.when(step > 0)
                def _():
                    scat_wait(pn2)
            else:
                scat_wait(pn2)

            if u == 0:
                idx_start(k + 2, pn2)
            else:
                @pl.when(step < NSTEP - 1)
                def _():
                    idx_start(k + 2, pn2)

            if u < 2:
                idx_wait(pn)
                gather_start(pn)
            else:
                @pl.when(step < NSTEP - 1)
                def _():
                    idx_wait(pn)
                    gather_start(pn)

            gather_wait(p)
            for g in range(BLK // L):
                sch = sv[p][pl.ds(g * L, L)]
                for i in range(L):
                    e = g * L + i
                    s = _splat(sch, i)
                    for c in range(D // L):
                        sl = pl.ds(c * L, L)
                        rv[p][e, sl] = rv[p][e, sl] * s
            scat_start(p)

    scat_wait((NBLK - 1) % 3)
    plsc.subcore_barrier()

    @pl.when(sid < 15)
    def _():
        pltpu.sync_copy(acc_sh.at[pl.ds(sid * RPS_A, RPS_A)],
                        acc_hbm.at[cid].at[pl.ds(sid * RPS_A, RPS_A)])

    @pl.when(sid == 15)
    def _():
        pltpu.sync_copy(acc_sh.at[pl.ds(15 * RPS_A, RPS_T)],
                        acc_hbm.at[cid].at[pl.ds(15 * RPS_A, RPS_T)])


BJ = 1000
NBJ = N_NODES // BJ


def _tc_layer(x, adds, w, root, b):
    combine = adds is not None

    def body(*refs):
        if combine:
            x_ref, a0_ref, a1_ref, w_ref, root_ref, b_ref, y_ref, o_ref = refs
        else:
            x_ref, w_ref, root_ref, b_ref, y_ref, o_ref = refs
        r = pl.program_id(1)
        xb = x_ref[...]
        if combine:
            xb = jnp.maximum(xb + a0_ref[...] + a1_ref[...], 0.0)
        y_ref[0] = lax.dot_general(xb, w_ref[0], (((1,), (0,)), ((), ())),
                                   precision=lax.Precision.HIGHEST)

        @pl.when(r == 0)
        def _():
            o_ref[...] = lax.dot_general(
                xb, root_ref[...], (((1,), (0,)), ((), ())),
                precision=lax.Precision.HIGHEST) + b_ref[...]

    x_spec = pl.BlockSpec((BJ, D), lambda j, r: (j, 0))
    in_specs = [x_spec]
    args = [x]
    if combine:
        in_specs += [x_spec, x_spec]
        args += [adds[0], adds[1]]
    in_specs += [
        pl.BlockSpec((1, D, D), lambda j, r: (r, 0, 0)),
        pl.BlockSpec((D, D), lambda j, r: (0, 0)),
        pl.BlockSpec((1, D), lambda j, r: (0, 0)),
    ]
    args += [w, root, b.reshape(1, D)]
    return pl.pallas_call(
        body,
        grid=(NBJ, N_R),
        in_specs=in_specs,
        out_specs=[
            pl.BlockSpec((1, BJ, D), lambda j, r: (r, j, 0)),
            pl.BlockSpec((BJ, D), lambda j, r: (j, 0)),
        ],
        out_shape=[
            jax.ShapeDtypeStruct((N_R, N_NODES, D), jnp.float32),
            jax.ShapeDtypeStruct((N_NODES, D), jnp.float32),
        ],
    )(*args)


def _tc_combine(o, a0, a1):
    def body(o_ref, a0_ref, a1_ref, out_ref):
        out_ref[...] = o_ref[...] + a0_ref[...] + a1_ref[...]

    spec = pl.BlockSpec((BJ, D), lambda j: (j, 0))
    return pl.pallas_call(
        body,
        grid=(NBJ,),
        in_specs=[spec, spec, spec],
        out_specs=spec,
        out_shape=jax.ShapeDtypeStruct((N_NODES, D), jnp.float32),
    )(o, a0, a1)


def kernel(edge_index, edge_type, emb, w1, root1, b1, w2, root2, b2):
    src = edge_index[0]
    dst = edge_index[1]
    pad = EP - E
    srcp = jnp.concatenate([src, jnp.zeros((pad,), jnp.int32)])
    dstp = jnp.concatenate([dst, jnp.full((pad,), PAD_DST, jnp.int32)])
    tp = jnp.concatenate([edge_type, jnp.zeros((pad,), jnp.int32)])

    zc = jnp.zeros((CN,), jnp.float32)
    cnt = _k_count(dstp, tp, zc)
    g, scale = _k_scale(srcp, dstp, tp, cnt)
    zacc = jnp.zeros((NODES_P, D), jnp.float32)

    y1, o1 = _tc_layer(emb, None, w1, root1, b1)
    a1 = _k_agg(y1.reshape(N_R * N_NODES, D), g, dstp, scale, zacc)
    y2, o2 = _tc_layer(o1, (a1[0, :N_NODES], a1[1, :N_NODES]), w2, root2, b2)
    a2 = _k_agg(y2.reshape(N_R * N_NODES, D), g, dstp, scale, zacc)
    return _tc_combine(o2, a2[0, :N_NODES], a2[1, :N_NODES])


# speedup vs baseline: 1.9305x; 1.9305x over previous
import functools

import jax
import jax.numpy as jnp
from jax import lax
from jax.experimental import pallas as pl
from jax.experimental.pallas import tpu as pltpu
from jax.experimental.pallas import tpu_sc as plsc

N_NODES = 10000
N_R = 16
D = 128
E = 320000

NC = 2
NS = 16
L = 16
NW = NC * NS

PAD_DST = N_NODES
NODES_P = 10016
RPS_A = 632
RPS_T = NODES_P - 15 * RPS_A
CN = 10112 * N_R
CNS = CN // NS
BLK = 128
NBLK = 81
NSTEP = NBLK // 3
EPW = NBLK * BLK
EP = EPW * NW

_mesh = plsc.VectorSubcoreMesh(core_axis_name="c", subcore_axis_name="s")

_GDN = lax.GatherDimensionNumbers(
    offset_dims=(), collapsed_slice_dims=(0,), start_index_map=(0,))


def _dg(v, idx):
    return lax.gather(v, idx[:, None], _GDN, (1,),
                      mode=lax.GatherScatterMode.PROMISE_IN_BOUNDS)


def _splat(v, i):
    return _dg(v, jnp.full((L,), i, jnp.int32))


@functools.partial(
    pl.kernel,
    out_type=jax.ShapeDtypeStruct((NC * CN,), jnp.float32),
    mesh=_mesh,
    scratch_types=[
        pltpu.VMEM((BLK,), jnp.int32),
        pltpu.VMEM((BLK,), jnp.int32),
        pltpu.VMEM((BLK,), jnp.int32),
        pltpu.VMEM((BLK,), jnp.float32),
        pltpu.VMEM_SHARED((CN,), jnp.float32),
    ],
)
def _k_count(dst_hbm, t_hbm, zc_hbm, cnt_hbm, dst_v, t_v, idx_v, ones_v, cnt_sh):
    cid = lax.axis_index("c")
    sid = lax.axis_index("s")
    wid = sid * NC + cid

    pltpu.sync_copy(zc_hbm.at[pl.ds(sid * CNS, CNS)],
                    cnt_sh.at[pl.ds(sid * CNS, CNS)])
    plsc.subcore_barrier()

    base = wid * EPW

    @pl.loop(0, NBLK)
    def _(blk):
        off = base + blk * BLK
        pltpu.sync_copy(t_hbm.at[pl.ds(off, BLK)], t_v)
        pltpu.sync_copy(dst_hbm.at[pl.ds(off, BLK)], dst_v)
        for q in range(BLK // L):
            sl = pl.ds(q * L, L)
            idx_v[sl] = dst_v[sl] * N_R + t_v[sl]
            pos = lax.iota(jnp.int32, L) + (off + q * L)
            ones_v[sl] = jnp.where(pos < E, 1.0, 0.0)
        pltpu.sync_copy(ones_v, cnt_sh.at[idx_v], add=True)

    plsc.subcore_barrier()
    pltpu.sync_copy(cnt_sh.at[pl.ds(sid * CNS, CNS)],
                    cnt_hbm.at[pl.ds(cid * CN + sid * CNS, CNS)])


@functools.partial(
    pl.kernel,
    out_type=[jax.ShapeDtypeStruct((EP,), jnp.int32),
              jax.ShapeDtypeStruct((EP,), jnp.float32)],
    mesh=_mesh,
    scratch_types=[
        pltpu.VMEM((BLK,), jnp.int32),
        pltpu.VMEM((BLK,), jnp.int32),
        pltpu.VMEM((BLK,), jnp.int32),
        pltpu.VMEM((BLK,), jnp.int32),
        pltpu.VMEM((BLK,), jnp.int32),
        pltpu.VMEM((BLK,), jnp.int32),
        pltpu.VMEM((BLK,), jnp.float32),
        pltpu.VMEM((BLK,), jnp.float32),
        pltpu.VMEM((BLK,), jnp.float32),
        pltpu.SemaphoreType.DMA,
        pltpu.SemaphoreType.DMA,
    ],
)
def _k_scale(src_hbm, dst_hbm, t_hbm, cnt_hbm, g_hbm, sc_hbm,
             src_v, dst_v, t_v, g_v, i0_v, i1_v, c0_v, c1_v, sc_v, sem0, sem1):
    cid = lax.axis_index("c")
    sid = lax.axis_index("s")
    wid = sid * NC + cid
    base = wid * EPW

    @pl.loop(0, NBLK)
    def _(blk):
        off = base + blk * BLK
        pltpu.sync_copy(src_hbm.at[pl.ds(off, BLK)], src_v)
        pltpu.sync_copy(dst_hbm.at[pl.ds(off, BLK)], dst_v)
        pltpu.sync_copy(t_hbm.at[pl.ds(off, BLK)], t_v)
        for q in range(BLK // L):
            sl = pl.ds(q * L, L)
            i0 = dst_v[sl] * N_R + t_v[sl]
            i0_v[sl] = i0
            i1_v[sl] = i0 + CN
            g_v[sl] = t_v[sl] * N_NODES + src_v[sl]
        cp0 = pltpu.async_copy(cnt_hbm.at[i0_v], c0_v, sem0)
        cp1 = pltpu.async_copy(cnt_hbm.at[i1_v], c1_v, sem1)
        cp0.wait()
        cp1.wait()
        for q in range(BLK // L):
            sl = pl.ds(q * L, L)
            pos = lax.iota(jnp.int32, L) + (off + q * L)
            s = 1.0 / jnp.maximum(c0_v[sl] + c1_v[sl], 1.0)
            sc_v[sl] = jnp.where(pos < E, s, 0.0)
        pltpu.sync_copy(g_v, g_hbm.at[pl.ds(off, BLK)])
        pltpu.sync_copy(sc_v, sc_hbm.at[pl.ds(off, BLK)])


@functools.partial(
    pl.kernel,
    out_type=jax.ShapeDtypeStruct((NC, NODES_P, D), jnp.float32),
    mesh=_mesh,
    scratch_types=(
        [pltpu.VMEM((BLK,), jnp.int32)] * 3
        + [pltpu.VMEM((BLK,), jnp.int32)] * 3
        + [pltpu.VMEM((BLK,), jnp.float32)] * 3
        + [pltpu.VMEM((BLK, D), jnp.float32)] * 3
        + [pltpu.VMEM_SHARED((NODES_P, D), jnp.float32)]
        + [pltpu.SemaphoreType.DMA] * 9
    ),
)
def _k_agg(y_hbm, g_hbm, dst_hbm, sc_hbm, z_hbm, acc_hbm,
           g0, g1, g2, d0, d1, d2, s0, s1, s2, r0, r1, r2, acc_sh,
           si0, si1, si2, sr0, sr1, sr2, ss0, ss1, ss2):
    gv = [g0, g1, g2]
    dv = [d0, d1, d2]
    sv = [s0, s1, s2]
    rv = [r0, r1, r2]
    sem_i = [si0, si1, si2]
    sem_r = [sr0, sr1, sr2]
    sem_s = [ss0, ss1, ss2]

    cid = lax.axis_index("c")
    sid = lax.axis_index("s")
    wid = sid * NC + cid
    base = wid * EPW

    def idx_start(k, p):
        off = base + k * BLK
        pltpu.async_copy(g_hbm.at[pl.ds(off, BLK)], gv[p], sem_i[p])
        pltpu.async_copy(dst_hbm.at[pl.ds(off, BLK)], dv[p], sem_i[p])
        pltpu.async_copy(sc_hbm.at[pl.ds(off, BLK)], sv[p], sem_i[p])

    def idx_wait(p):
        pltpu.make_async_copy(g_hbm.at[pl.ds(0, BLK)], gv[p], sem_i[p]).wait()
        pltpu.make_async_copy(dst_hbm.at[pl.ds(0, BLK)], dv[p], sem_i[p]).wait()
        pltpu.make_async_copy(sc_hbm.at[pl.ds(0, BLK)], sv[p], sem_i[p]).wait()

    def gather_start(p):
        pltpu.async_copy(y_hbm.at[gv[p]], rv[p], sem_r[p])

    def gather_wait(p):
        pltpu.make_async_copy(y_hbm.at[gv[p]], rv[p], sem_r[p]).wait()

    def scat_start(p):
        pltpu.async_copy(rv[p], acc_sh.at[dv[p]], sem_s[p], add=True)

    def scat_wait(p):
        pltpu.make_async_copy(rv[p], acc_sh.at[dv[p]], sem_s[p]).wait()

    @pl.when(sid < 15)
    def _():
        pltpu.sync_copy(z_hbm.at[pl.ds(sid * RPS_A, RPS_A)],
                        acc_sh.at[pl.ds(sid * RPS_A, RPS_A)])

    @pl.when(sid == 15)
    def _():
        pltpu.sync_copy(z_hbm.at[pl.ds(15 * RPS_A, RPS_T)],
                        acc_sh.at[pl.ds(15 * RPS_A, RPS_T)])

    plsc.subcore_barrier()

    idx_start(0, 0)
    idx_start(1, 1)
    idx_wait(0)
    gather_start(0)

    @pl.loop(0, NSTEP)
    def _(step):
        for u in range(3):
            k = step * 3 + u
            p, pn, pn2 = u, (u + 1) % 3, (u + 2) % 3

            if u == 0:
                @pl.when(step > 0)
                def _():
                    scat_wait(pn2)
            else:
                scat_wait(pn2)

            if u == 0:
                idx_start(k + 2, pn2)
            else:
                @pl.when(step < NSTEP - 1)
                def _():
                    idx_start(k + 2, pn2)

            if u < 2:
                idx_wait(pn)
                gather_start(pn)
            else:
                @pl.when(step < NSTEP - 1)
                def _():
                    idx_wait(pn)
                    gather_start(pn)

            gather_wait(p)
            for g in range(BLK // L):
                sch = sv[p][pl.ds(g * L, L)]
                for i in range(L):
                    e = g * L + i
                    s = _splat(sch, i)
                    for c in range(D // L):
                        sl = pl.ds(c * L, L)
                        rv[p][e, sl] = rv[p][e, sl] * s
            scat_start(p)

    scat_wait((NBLK - 1) % 3)
    plsc.subcore_barrier()

    @pl.when(sid < 15)
    def _():
        pltpu.sync_copy(acc_sh.at[pl.ds(sid * RPS_A, RPS_A)],
                        acc_hbm.at[cid].at[pl.ds(sid * RPS_A, RPS_A)])

    @pl.when(sid == 15)
    def _():
        pltpu.sync_copy(acc_sh.at[pl.ds(15 * RPS_A, RPS_T)],
                        acc_hbm.at[cid].at[pl.ds(15 * RPS_A, RPS_T)])


BJ = 1000
NBJ = N_NODES // BJ


def _tc_layer(x, adds, w, root, b):
    combine = adds is not None

    def body(*refs):
        if combine:
            x_ref, a0_ref, a1_ref, w_ref, root_ref, b_ref, y_ref, o_ref = refs
        else:
            x_ref, w_ref, root_ref, b_ref, y_ref, o_ref = refs
        r = pl.program_id(1)
        xb = x_ref[...]
        if combine:
            xb = jnp.maximum(xb + a0_ref[...] + a1_ref[...], 0.0)
        y_ref[0] = lax.dot_general(xb, w_ref[0], (((1,), (0,)), ((), ())),
                                   precision=lax.Precision.HIGHEST)

        @pl.when(r == 0)
        def _():
            o_ref[...] = lax.dot_general(
                xb, root_ref[...], (((1,), (0,)), ((), ())),
                precision=lax.Precision.HIGHEST) + b_ref[...]

    x_spec = pl.BlockSpec((BJ, D), lambda j, r: (j, 0))
    in_specs = [x_spec]
    args = [x]
    if combine:
        in_specs += [x_spec, x_spec]
        args += [adds[0], adds[1]]
    in_specs += [
        pl.BlockSpec((1, D, D), lambda j, r: (r, 0, 0)),
        pl.BlockSpec((D, D), lambda j, r: (0, 0)),
        pl.BlockSpec((1, D), lambda j, r: (0, 0)),
    ]
    args += [w, root, b.reshape(1, D)]
    return pl.pallas_call(
        body,
        grid=(NBJ, N_R),
        in_specs=in_specs,
        out_specs=[
            pl.BlockSpec((1, BJ, D), lambda j, r: (r, j, 0)),
            pl.BlockSpec((BJ, D), lambda j, r: (j, 0)),
        ],
        out_shape=[
            jax.ShapeDtypeStruct((N_R, N_NODES, D), jnp.float32),
            jax.ShapeDtypeStruct((N_NODES, D), jnp.float32),
        ],
    )(*args)


def _tc_combine(o, a0, a1):
    def body(o_ref, a0_ref, a1_ref, out_ref):
        out_ref[...] = o_ref[...] + a0_ref[...] + a1_ref[...]

    spec = pl.BlockSpec((BJ, D), lambda j: (j, 0))
    return pl.pallas_call(
        body,
        grid=(NBJ,),
        in_specs=[spec, spec, spec],
        out_specs=spec,
        out_shape=jax.ShapeDtypeStruct((N_NODES, D), jnp.float32),
    )(o, a0, a1)


def kernel(edge_index, edge_type, emb, w1, root1, b1, w2, root2, b2):
    src = edge_index[0]
    dst = edge_index[1]
    pad = EP - E
    spread = (jnp.arange(pad, dtype=jnp.int32) * 37) % N_NODES
    srcp = jnp.concatenate([src, spread])
    dstp = jnp.concatenate([dst, spread])
    tp = jnp.concatenate([edge_type, jnp.zeros((pad,), jnp.int32)])

    zc = jnp.zeros((CN,), jnp.float32)
    cnt = _k_count(dstp, tp, zc)
    g, scale = _k_scale(srcp, dstp, tp, cnt)
    zacc = jnp.zeros((NODES_P, D), jnp.float32)

    y1, o1 = _tc_layer(emb, None, w1, root1, b1)
    a1 = _k_agg(y1.reshape(N_R * N_NODES, D), g, dstp, scale, zacc)
    y2, o2 = _tc_layer(o1, (a1[0, :N_NODES], a1[1, :N_NODES]), w2, root2, b2)
    a2 = _k_agg(y2.reshape(N_R * N_NODES, D), g, dstp, scale, zacc)
    return _tc_combine(o2, a2[0, :N_NODES], a2[1, :N_NODES])


# speedup vs baseline: 2.0428x; 1.0582x over previous
import functools

import jax
import jax.numpy as jnp
from jax import lax
from jax.experimental import pallas as pl
from jax.experimental.pallas import tpu as pltpu
from jax.experimental.pallas import tpu_sc as plsc

N_NODES = 10000
N_R = 16
D = 128
E = 320000

NC = 2
NS = 16
L = 16
NW = NC * NS

PAD_DST = N_NODES
NODES_P = 10016
RPS_A = 632
RPS_T = NODES_P - 15 * RPS_A
CN = 10112 * N_R
CNS = CN // NS
BLK = 128
NBLK = 81
NSTEP = NBLK // 3
EPW = NBLK * BLK
EP = EPW * NW

_mesh = plsc.VectorSubcoreMesh(core_axis_name="c", subcore_axis_name="s")

_GDN = lax.GatherDimensionNumbers(
    offset_dims=(), collapsed_slice_dims=(0,), start_index_map=(0,))


def _dg(v, idx):
    return lax.gather(v, idx[:, None], _GDN, (1,),
                      mode=lax.GatherScatterMode.PROMISE_IN_BOUNDS)


def _splat(v, i):
    return _dg(v, jnp.full((L,), i, jnp.int32))


@functools.partial(
    pl.kernel,
    out_type=jax.ShapeDtypeStruct((NC * CN,), jnp.float32),
    mesh=_mesh,
    scratch_types=[
        pltpu.VMEM((BLK,), jnp.int32),
        pltpu.VMEM((BLK,), jnp.int32),
        pltpu.VMEM((BLK,), jnp.int32),
        pltpu.VMEM((BLK,), jnp.float32),
        pltpu.VMEM_SHARED((CN,), jnp.float32),
    ],
)
def _k_count(dst_hbm, t_hbm, zc_hbm, cnt_hbm, dst_v, t_v, idx_v, ones_v, cnt_sh):
    cid = lax.axis_index("c")
    sid = lax.axis_index("s")
    wid = sid * NC + cid

    pltpu.sync_copy(zc_hbm.at[pl.ds(sid * CNS, CNS)],
                    cnt_sh.at[pl.ds(sid * CNS, CNS)])
    plsc.subcore_barrier()

    base = wid * EPW

    @pl.loop(0, NBLK)
    def _(blk):
        off = base + blk * BLK
        pltpu.sync_copy(t_hbm.at[pl.ds(off, BLK)], t_v)
        pltpu.sync_copy(dst_hbm.at[pl.ds(off, BLK)], dst_v)
        for q in range(BLK // L):
            sl = pl.ds(q * L, L)
            idx_v[sl] = dst_v[sl] * N_R + t_v[sl]
            pos = lax.iota(jnp.int32, L) + (off + q * L)
            ones_v[sl] = jnp.where(pos < E, 1.0, 0.0)
        pltpu.sync_copy(ones_v, cnt_sh.at[idx_v], add=True)

    plsc.subcore_barrier()
    pltpu.sync_copy(cnt_sh.at[pl.ds(sid * CNS, CNS)],
                    cnt_hbm.at[pl.ds(cid * CN + sid * CNS, CNS)])


@functools.partial(
    pl.kernel,
    out_type=[jax.ShapeDtypeStruct((EP,), jnp.int32),
              jax.ShapeDtypeStruct((EP,), jnp.float32)],
    mesh=_mesh,
    scratch_types=[
        pltpu.VMEM((BLK,), jnp.int32),
        pltpu.VMEM((BLK,), jnp.int32),
        pltpu.VMEM((BLK,), jnp.int32),
        pltpu.VMEM((BLK,), jnp.int32),
        pltpu.VMEM((BLK,), jnp.int32),
        pltpu.VMEM((BLK,), jnp.int32),
        pltpu.VMEM((BLK,), jnp.float32),
        pltpu.VMEM((BLK,), jnp.float32),
        pltpu.VMEM((BLK,), jnp.float32),
        pltpu.SemaphoreType.DMA,
        pltpu.SemaphoreType.DMA,
    ],
)
def _k_scale(src_hbm, dst_hbm, t_hbm, cnt_hbm, g_hbm, sc_hbm,
             src_v, dst_v, t_v, g_v, i0_v, i1_v, c0_v, c1_v, sc_v, sem0, sem1):
    cid = lax.axis_index("c")
    sid = lax.axis_index("s")
    wid = sid * NC + cid
    base = wid * EPW

    @pl.loop(0, NBLK)
    def _(blk):
        off = base + blk * BLK
        pltpu.sync_copy(src_hbm.at[pl.ds(off, BLK)], src_v)
        pltpu.sync_copy(dst_hbm.at[pl.ds(off, BLK)], dst_v)
        pltpu.sync_copy(t_hbm.at[pl.ds(off, BLK)], t_v)
        for q in range(BLK // L):
            sl = pl.ds(q * L, L)
            i0 = dst_v[sl] * N_R + t_v[sl]
            i0_v[sl] = i0
            i1_v[sl] = i0 + CN
            g_v[sl] = t_v[sl] * N_NODES + src_v[sl]
        cp0 = pltpu.async_copy(cnt_hbm.at[i0_v], c0_v, sem0)
        cp1 = pltpu.async_copy(cnt_hbm.at[i1_v], c1_v, sem1)
        cp0.wait()
        cp1.wait()
        for q in range(BLK // L):
            sl = pl.ds(q * L, L)
            pos = lax.iota(jnp.int32, L) + (off + q * L)
            s = 1.0 / jnp.maximum(c0_v[sl] + c1_v[sl], 1.0)
            sc_v[sl] = jnp.where(pos < E, s, 0.0)
        pltpu.sync_copy(g_v, g_hbm.at[pl.ds(off, BLK)])
        pltpu.sync_copy(sc_v, sc_hbm.at[pl.ds(off, BLK)])


@functools.partial(
    pl.kernel,
    out_type=jax.ShapeDtypeStruct((NC, NODES_P, D), jnp.float32),
    mesh=_mesh,
    scratch_types=(
        [pltpu.VMEM((BLK,), jnp.int32)] * 3
        + [pltpu.VMEM((BLK,), jnp.int32)] * 3
        + [pltpu.VMEM((BLK,), jnp.float32)] * 3
        + [pltpu.VMEM((BLK, D), jnp.float32)] * 3
        + [pltpu.VMEM_SHARED((NODES_P, D), jnp.float32)]
        + [pltpu.SemaphoreType.DMA] * 9
    ),
)
def _k_agg(y_hbm, g_hbm, dst_hbm, sc_hbm, z_hbm, acc_hbm,
           g0, g1, g2, d0, d1, d2, s0, s1, s2, r0, r1, r2, acc_sh,
           si0, si1, si2, sr0, sr1, sr2, ss0, ss1, ss2):
    gv = [g0, g1, g2]
    dv = [d0, d1, d2]
    sv = [s0, s1, s2]
    rv = [r0, r1, r2]
    sem_i = [si0, si1, si2]
    sem_r = [sr0, sr1, sr2]
    sem_s = [ss0, ss1, ss2]

    cid = lax.axis_index("c")
    sid = lax.axis_index("s")
    wid = sid * NC + cid
    base = wid * EPW

    def idx_start(k, p):
        off = base + k * BLK
        pltpu.async_copy(g_hbm.at[pl.ds(off, BLK)], gv[p], sem_i[p])
        pltpu.async_copy(dst_hbm.at[pl.ds(off, BLK)], dv[p], sem_i[p])
        pltpu.async_copy(sc_hbm.at[pl.ds(off, BLK)], sv[p], sem_i[p])

    def idx_wait(p):
        pltpu.make_async_copy(g_hbm.at[pl.ds(0, BLK)], gv[p], sem_i[p]).wait()
        pltpu.make_async_copy(dst_hbm.at[pl.ds(0, BLK)], dv[p], sem_i[p]).wait()
        pltpu.make_async_copy(sc_hbm.at[pl.ds(0, BLK)], sv[p], sem_i[p]).wait()

    def gather_start(p):
        pltpu.async_copy(y_hbm.at[gv[p]], rv[p], sem_r[p])

    def gather_wait(p):
        pltpu.make_async_copy(y_hbm.at[gv[p]], rv[p], sem_r[p]).wait()

    def scat_start(p):
        pltpu.async_copy(rv[p], acc_sh.at[dv[p]], sem_s[p], add=True)

    def scat_wait(p):
        pltpu.make_async_copy(rv[p], acc_sh.at[dv[p]], sem_s[p]).wait()

    @pl.when(sid < 15)
    def _():
        pltpu.sync_copy(z_hbm.at[pl.ds(sid * RPS_A, RPS_A)],
                        acc_sh.at[pl.ds(sid * RPS_A, RPS_A)])

    @pl.when(sid == 15)
    def _():
        pltpu.sync_copy(z_hbm.at[pl.ds(15 * RPS_A, RPS_T)],
                        acc_sh.at[pl.ds(15 * RPS_A, RPS_T)])

    plsc.subcore_barrier()

    idx_start(0, 0)
    idx_start(1, 1)
    idx_wait(0)
    gather_start(0)

    @pl.loop(0, NSTEP)
    def _(step):
        for u in range(3):
            k = step * 3 + u
            p, pn, pn2 = u, (u + 1) % 3, (u + 2) % 3

            if u == 0:
                @pl.when(step > 0)
                def _():
                    scat_wait(pn2)
            else:
                scat_wait(pn2)

            if u == 0:
                idx_start(k + 2, pn2)
            else:
                @pl.when(step < NSTEP - 1)
                def _():
                    idx_start(k + 2, pn2)

            if u < 2:
                idx_wait(pn)
                gather_start(pn)
            else:
                @pl.when(step < NSTEP - 1)
                def _():
                    idx_wait(pn)
                    gather_start(pn)

            gather_wait(p)
            for g in range(BLK // L):
                sch = sv[p][pl.ds(g * L, L)]
                for i in range(L):
                    e = g * L + i
                    s = _splat(sch, i)
                    for c in range(D // L):
                        sl = pl.ds(c * L, L)
                        rv[p][e, sl] = rv[p][e, sl] * s
            scat_start(p)

    scat_wait((NBLK - 1) % 3)
    plsc.subcore_barrier()

    @pl.when(sid < 15)
    def _():
        pltpu.sync_copy(acc_sh.at[pl.ds(sid * RPS_A, RPS_A)],
                        acc_hbm.at[cid].at[pl.ds(sid * RPS_A, RPS_A)])

    @pl.when(sid == 15)
    def _():
        pltpu.sync_copy(acc_sh.at[pl.ds(15 * RPS_A, RPS_T)],
                        acc_hbm.at[cid].at[pl.ds(15 * RPS_A, RPS_T)])


BJ = 1000
NBJ = N_NODES // BJ


def _tc_layer(x, adds, w, root, b):
    combine = adds is not None

    def body(*refs):
        if combine:
            x_ref, a0_ref, a1_ref, w_ref, root_ref, b_ref, y_ref, o_ref = refs
        else:
            x_ref, w_ref, root_ref, b_ref, y_ref, o_ref = refs
        r = pl.program_id(1)
        xb = x_ref[...]
        if combine:
            xb = jnp.maximum(xb + a0_ref[...] + a1_ref[...], 0.0)
        y_ref[0] = lax.dot_general(
            xb.astype(jnp.bfloat16), w_ref[0].astype(jnp.bfloat16),
            (((1,), (0,)), ((), ())),
            preferred_element_type=jnp.float32)

        @pl.when(r == 0)
        def _():
            o_ref[...] = lax.dot_general(
                xb, root_ref[...], (((1,), (0,)), ((), ())),
                precision=lax.Precision.HIGHEST) + b_ref[...]

    x_spec = pl.BlockSpec((BJ, D), lambda j, r: (j, 0))
    in_specs = [x_spec]
    args = [x]
    if combine:
        in_specs += [x_spec, x_spec]
        args += [adds[0], adds[1]]
    in_specs += [
        pl.BlockSpec((1, D, D), lambda j, r: (r, 0, 0)),
        pl.BlockSpec((D, D), lambda j, r: (0, 0)),
        pl.BlockSpec((1, D), lambda j, r: (0, 0)),
    ]
    args += [w, root, b.reshape(1, D)]
    return pl.pallas_call(
        body,
        grid=(NBJ, N_R),
        in_specs=in_specs,
        out_specs=[
            pl.BlockSpec((1, BJ, D), lambda j, r: (r, j, 0)),
            pl.BlockSpec((BJ, D), lambda j, r: (j, 0)),
        ],
        out_shape=[
            jax.ShapeDtypeStruct((N_R, N_NODES, D), jnp.float32),
            jax.ShapeDtypeStruct((N_NODES, D), jnp.float32),
        ],
    )(*args)


def _tc_combine(o, a0, a1):
    def body(o_ref, a0_ref, a1_ref, out_ref):
        out_ref[...] = o_ref[...] + a0_ref[...] + a1_ref[...]

    spec = pl.BlockSpec((BJ, D), lambda j: (j, 0))
    return pl.pallas_call(
        body,
        grid=(NBJ,),
        in_specs=[spec, spec, spec],
        out_specs=spec,
        out_shape=jax.ShapeDtypeStruct((N_NODES, D), jnp.float32),
    )(o, a0, a1)


def kernel(edge_index, edge_type, emb, w1, root1, b1, w2, root2, b2):
    src = edge_index[0]
    dst = edge_index[1]
    pad = EP - E
    spread = (jnp.arange(pad, dtype=jnp.int32) * 37) % N_NODES
    srcp = jnp.concatenate([src, spread])
    dstp = jnp.concatenate([dst, spread])
    tp = jnp.concatenate([edge_type, jnp.zeros((pad,), jnp.int32)])

    zc = jnp.zeros((CN,), jnp.float32)
    cnt = _k_count(dstp, tp, zc)
    g, scale = _k_scale(srcp, dstp, tp, cnt)
    zacc = jnp.zeros((NODES_P, D), jnp.float32)

    y1, o1 = _tc_layer(emb, None, w1, root1, b1)
    a1 = _k_agg(y1.reshape(N_R * N_NODES, D), g, dstp, scale, zacc)
    y2, o2 = _tc_layer(o1, (a1[0, :N_NODES], a1[1, :N_NODES]), w2, root2, b2)
    a2 = _k_agg(y2.reshape(N_R * N_NODES, D), g, dstp, scale, zacc)
    return _tc_combine(o2, a2[0, :N_NODES], a2[1, :N_NODES])


# speedup vs baseline: 2.2344x; 1.0938x over previous
import functools

import jax
import jax.numpy as jnp
from jax import lax
from jax.experimental import pallas as pl
from jax.experimental.pallas import tpu as pltpu
from jax.experimental.pallas import tpu_sc as plsc

N_NODES = 10000
N_R = 16
D = 128
E = 320000

NC = 2
NS = 16
L = 16
NW = NC * NS

PAD_DST = N_NODES
NODES_P = 10016
RPS_A = 632
RPS_T = NODES_P - 15 * RPS_A
CN = 10112 * N_R
CNS = CN // NS
BLK = 128
NBLK = 81
NSTEP = NBLK // 3
EPW = NBLK * BLK
EP = EPW * NW

_mesh = plsc.VectorSubcoreMesh(core_axis_name="c", subcore_axis_name="s")

_GDN = lax.GatherDimensionNumbers(
    offset_dims=(), collapsed_slice_dims=(0,), start_index_map=(0,))


def _dg(v, idx):
    return lax.gather(v, idx[:, None], _GDN, (1,),
                      mode=lax.GatherScatterMode.PROMISE_IN_BOUNDS)


def _splat(v, i):
    return _dg(v, jnp.full((L,), i, jnp.int32))


@functools.partial(
    pl.kernel,
    out_type=jax.ShapeDtypeStruct((NC * CN,), jnp.float32),
    mesh=_mesh,
    scratch_types=[
        pltpu.VMEM((BLK,), jnp.int32),
        pltpu.VMEM((BLK,), jnp.int32),
        pltpu.VMEM((BLK,), jnp.int32),
        pltpu.VMEM((BLK,), jnp.float32),
        pltpu.VMEM_SHARED((CN,), jnp.float32),
    ],
)
def _k_count(dst_hbm, t_hbm, zc_hbm, cnt_hbm, dst_v, t_v, idx_v, ones_v, cnt_sh):
    cid = lax.axis_index("c")
    sid = lax.axis_index("s")
    wid = sid * NC + cid

    pltpu.sync_copy(zc_hbm.at[pl.ds(sid * CNS, CNS)],
                    cnt_sh.at[pl.ds(sid * CNS, CNS)])
    plsc.subcore_barrier()

    base = wid * EPW

    @pl.loop(0, NBLK)
    def _(blk):
        off = base + blk * BLK
        pltpu.sync_copy(t_hbm.at[pl.ds(off, BLK)], t_v)
        pltpu.sync_copy(dst_hbm.at[pl.ds(off, BLK)], dst_v)
        for q in range(BLK // L):
            sl = pl.ds(q * L, L)
            idx_v[sl] = dst_v[sl] * N_R + t_v[sl]
            pos = lax.iota(jnp.int32, L) + (off + q * L)
            ones_v[sl] = jnp.where(pos < E, 1.0, 0.0)
        pltpu.sync_copy(ones_v, cnt_sh.at[idx_v], add=True)

    plsc.subcore_barrier()
    pltpu.sync_copy(cnt_sh.at[pl.ds(sid * CNS, CNS)],
                    cnt_hbm.at[pl.ds(cid * CN + sid * CNS, CNS)])


@functools.partial(
    pl.kernel,
    out_type=[jax.ShapeDtypeStruct((EP,), jnp.int32),
              jax.ShapeDtypeStruct((EP,), jnp.float32)],
    mesh=_mesh,
    scratch_types=(
        [pltpu.VMEM((BLK,), jnp.int32)] * 9
        + [pltpu.VMEM((BLK,), jnp.int32)] * 9
        + [pltpu.VMEM((BLK,), jnp.float32)] * 9
        + [pltpu.SemaphoreType.DMA] * 9
    ),
)
def _k_scale(src_hbm, dst_hbm, t_hbm, cnt_hbm, g_hbm, sc_hbm,
             sr0, sr1, sr2, dd0, dd1, dd2, tt0, tt1, tt2,
             gg0, gg1, gg2, ia0, ia1, ia2, ib0, ib1, ib2,
             c00, c01, c02, c10, c11, c12, sc0, sc1, sc2,
             si0, si1, si2, sg0, sg1, sg2, sw0, sw1, sw2):
    srv = [sr0, sr1, sr2]
    dv = [dd0, dd1, dd2]
    tv = [tt0, tt1, tt2]
    gv = [gg0, gg1, gg2]
    i0v = [ia0, ia1, ia2]
    i1v = [ib0, ib1, ib2]
    c0v = [c00, c01, c02]
    c1v = [c10, c11, c12]
    scv = [sc0, sc1, sc2]
    sem_i = [si0, si1, si2]
    sem_g = [sg0, sg1, sg2]
    sem_w = [sw0, sw1, sw2]

    cid = lax.axis_index("c")
    sid = lax.axis_index("s")
    wid = sid * NC + cid
    base = wid * EPW

    def idx_start(k, p):
        off = base + k * BLK
        pltpu.async_copy(src_hbm.at[pl.ds(off, BLK)], srv[p], sem_i[p])
        pltpu.async_copy(dst_hbm.at[pl.ds(off, BLK)], dv[p], sem_i[p])
        pltpu.async_copy(t_hbm.at[pl.ds(off, BLK)], tv[p], sem_i[p])

    def stage_b(p):
        pltpu.make_async_copy(src_hbm.at[pl.ds(0, BLK)], srv[p], sem_i[p]).wait()
        pltpu.make_async_copy(dst_hbm.at[pl.ds(0, BLK)], dv[p], sem_i[p]).wait()
        pltpu.make_async_copy(t_hbm.at[pl.ds(0, BLK)], tv[p], sem_i[p]).wait()
        for q in range(BLK // L):
            sl = pl.ds(q * L, L)
            i0 = dv[p][sl] * N_R + tv[p][sl]
            i0v[p][sl] = i0
            i1v[p][sl] = i0 + CN
            gv[p][sl] = tv[p][sl] * N_NODES + srv[p][sl]
        pltpu.async_copy(cnt_hbm.at[i0v[p]], c0v[p], sem_g[p])
        pltpu.async_copy(cnt_hbm.at[i1v[p]], c1v[p], sem_g[p])

    def stage_c(k, p):
        off = base + k * BLK
        pltpu.make_async_copy(cnt_hbm.at[i0v[p]], c0v[p], sem_g[p]).wait()
        pltpu.make_async_copy(cnt_hbm.at[i1v[p]], c1v[p], sem_g[p]).wait()
        for q in range(BLK // L):
            sl = pl.ds(q * L, L)
            pos = lax.iota(jnp.int32, L) + (off + q * L)
            s = 1.0 / jnp.maximum(c0v[p][sl] + c1v[p][sl], 1.0)
            scv[p][sl] = jnp.where(pos < E, s, 0.0)
        pltpu.async_copy(gv[p], g_hbm.at[pl.ds(off, BLK)], sem_w[p])
        pltpu.async_copy(scv[p], sc_hbm.at[pl.ds(off, BLK)], sem_w[p])

    def w_wait(p):
        pltpu.make_async_copy(gv[p], g_hbm.at[pl.ds(0, BLK)], sem_w[p]).wait()
        pltpu.make_async_copy(scv[p], sc_hbm.at[pl.ds(0, BLK)], sem_w[p]).wait()

    idx_start(0, 0)
    idx_start(1, 1)
    stage_b(0)

    @pl.loop(0, NSTEP)
    def _(step):
        for u in range(3):
            k = step * 3 + u
            p, pn, pn2 = u, (u + 1) % 3, (u + 2) % 3

            if u == 0:
                @pl.when(step > 0)
                def _():
                    w_wait(pn2)
            else:
                w_wait(pn2)

            if u == 0:
                idx_start(k + 2, pn2)
            else:
                @pl.when(step < NSTEP - 1)
                def _():
                    idx_start(k + 2, pn2)

            if u < 2:
                stage_b(pn)
            else:
                @pl.when(step < NSTEP - 1)
                def _():
                    stage_b(pn)

            stage_c(k, p)

    w_wait((NBLK - 1) % 3)


@functools.partial(
    pl.kernel,
    out_type=jax.ShapeDtypeStruct((NC, NODES_P, D), jnp.float32),
    mesh=_mesh,
    scratch_types=(
        [pltpu.VMEM((BLK,), jnp.int32)] * 3
        + [pltpu.VMEM((BLK,), jnp.int32)] * 3
        + [pltpu.VMEM((BLK,), jnp.float32)] * 3
        + [pltpu.VMEM((BLK, D), jnp.float32)] * 3
        + [pltpu.VMEM_SHARED((NODES_P, D), jnp.float32)]
        + [pltpu.SemaphoreType.DMA] * 9
    ),
)
def _k_agg(y_hbm, g_hbm, dst_hbm, sc_hbm, z_hbm, acc_hbm,
           g0, g1, g2, d0, d1, d2, s0, s1, s2, r0, r1, r2, acc_sh,
           si0, si1, si2, sr0, sr1, sr2, ss0, ss1, ss2):
    gv = [g0, g1, g2]
    dv = [d0, d1, d2]
    sv = [s0, s1, s2]
    rv = [r0, r1, r2]
    sem_i = [si0, si1, si2]
    sem_r = [sr0, sr1, sr2]
    sem_s = [ss0, ss1, ss2]

    cid = lax.axis_index("c")
    sid = lax.axis_index("s")
    wid = sid * NC + cid
    base = wid * EPW

    def idx_start(k, p):
        off = base + k * BLK
        pltpu.async_copy(g_hbm.at[pl.ds(off, BLK)], gv[p], sem_i[p])
        pltpu.async_copy(dst_hbm.at[pl.ds(off, BLK)], dv[p], sem_i[p])
        pltpu.async_copy(sc_hbm.at[pl.ds(off, BLK)], sv[p], sem_i[p])

    def idx_wait(p):
        pltpu.make_async_copy(g_hbm.at[pl.ds(0, BLK)], gv[p], sem_i[p]).wait()
        pltpu.make_async_copy(dst_hbm.at[pl.ds(0, BLK)], dv[p], sem_i[p]).wait()
        pltpu.make_async_copy(sc_hbm.at[pl.ds(0, BLK)], sv[p], sem_i[p]).wait()

    def gather_start(p):
        pltpu.async_copy(y_hbm.at[gv[p]], rv[p], sem_r[p])

    def gather_wait(p):
        pltpu.make_async_copy(y_hbm.at[gv[p]], rv[p], sem_r[p]).wait()

    def scat_start(p):
        pltpu.async_copy(rv[p], acc_sh.at[dv[p]], sem_s[p], add=True)

    def scat_wait(p):
        pltpu.make_async_copy(rv[p], acc_sh.at[dv[p]], sem_s[p]).wait()

    @pl.when(sid < 15)
    def _():
        pltpu.sync_copy(z_hbm.at[pl.ds(sid * RPS_A, RPS_A)],
                        acc_sh.at[pl.ds(sid * RPS_A, RPS_A)])

    @pl.when(sid == 15)
    def _():
        pltpu.sync_copy(z_hbm.at[pl.ds(15 * RPS_A, RPS_T)],
                        acc_sh.at[pl.ds(15 * RPS_A, RPS_T)])

    plsc.subcore_barrier()

    idx_start(0, 0)
    idx_start(1, 1)
    idx_wait(0)
    gather_start(0)

    @pl.loop(0, NSTEP)
    def _(step):
        for u in range(3):
            k = step * 3 + u
            p, pn, pn2 = u, (u + 1) % 3, (u + 2) % 3

            if u == 0:
                @pl.when(step > 0)
                def _():
                    scat_wait(pn2)
            else:
                scat_wait(pn2)

            if u == 0:
                idx_start(k + 2, pn2)
            else:
                @pl.when(step < NSTEP - 1)
                def _():
                    idx_start(k + 2, pn2)

            if u < 2:
                idx_wait(pn)
                gather_start(pn)
            else:
                @pl.when(step < NSTEP - 1)
                def _():
                    idx_wait(pn)
                    gather_start(pn)

            gather_wait(p)
            for g in range(BLK // L):
                sch = sv[p][pl.ds(g * L, L)]
                for i in range(L):
                    e = g * L + i
                    s = _splat(sch, i)
                    for c in range(D // L):
                        sl = pl.ds(c * L, L)
                        rv[p][e, sl] = rv[p][e, sl] * s
            scat_start(p)

    scat_wait((NBLK - 1) % 3)
    plsc.subcore_barrier()

    @pl.when(sid < 15)
    def _():
        pltpu.sync_copy(acc_sh.at[pl.ds(sid * RPS_A, RPS_A)],
                        acc_hbm.at[cid].at[pl.ds(sid * RPS_A, RPS_A)])

    @pl.when(sid == 15)
    def _():
        pltpu.sync_copy(acc_sh.at[pl.ds(15 * RPS_A, RPS_T)],
                        acc_hbm.at[cid].at[pl.ds(15 * RPS_A, RPS_T)])


BJ = 1000
NBJ = N_NODES // BJ


def _tc_layer(x, adds, w, root, b):
    combine = adds is not None

    def body(*refs):
        if combine:
            x_ref, a0_ref, a1_ref, w_ref, root_ref, b_ref, y_ref, o_ref = refs
        else:
            x_ref, w_ref, root_ref, b_ref, y_ref, o_ref = refs
        r = pl.program_id(1)
        xb = x_ref[...]
        if combine:
            xb = jnp.maximum(xb + a0_ref[...] + a1_ref[...], 0.0)
        y_ref[0] = lax.dot_general(
            xb.astype(jnp.bfloat16), w_ref[0].astype(jnp.bfloat16),
            (((1,), (0,)), ((), ())),
            preferred_element_type=jnp.float32)

        @pl.when(r == 0)
        def _():
            o_ref[...] = lax.dot_general(
                xb, root_ref[...], (((1,), (0,)), ((), ())),
                precision=lax.Precision.HIGHEST) + b_ref[...]

    x_spec = pl.BlockSpec((BJ, D), lambda j, r: (j, 0))
    in_specs = [x_spec]
    args = [x]
    if combine:
        in_specs += [x_spec, x_spec]
        args += [adds[0], adds[1]]
    in_specs += [
        pl.BlockSpec((1, D, D), lambda j, r: (r, 0, 0)),
        pl.BlockSpec((D, D), lambda j, r: (0, 0)),
        pl.BlockSpec((1, D), lambda j, r: (0, 0)),
    ]
    args += [w, root, b.reshape(1, D)]
    return pl.pallas_call(
        body,
        grid=(NBJ, N_R),
        in_specs=in_specs,
        out_specs=[
            pl.BlockSpec((1, BJ, D), lambda j, r: (r, j, 0)),
            pl.BlockSpec((BJ, D), lambda j, r: (j, 0)),
        ],
        out_shape=[
            jax.ShapeDtypeStruct((N_R, N_NODES, D), jnp.float32),
            jax.ShapeDtypeStruct((N_NODES, D), jnp.float32),
        ],
    )(*args)


def _tc_combine(o, a0, a1):
    def body(o_ref, a0_ref, a1_ref, out_ref):
        out_ref[...] = o_ref[...] + a0_ref[...] + a1_ref[...]

    spec = pl.BlockSpec((BJ, D), lambda j: (j, 0))
    return pl.pallas_call(
        body,
        grid=(NBJ,),
        in_specs=[spec, spec, spec],
        out_specs=spec,
        out_shape=jax.ShapeDtypeStruct((N_NODES, D), jnp.float32),
    )(o, a0, a1)


def kernel(edge_index, edge_type, emb, w1, root1, b1, w2, root2, b2):
    src = edge_index[0]
    dst = edge_index[1]
    pad = EP - E
    spread = (jnp.arange(pad, dtype=jnp.int32) * 37) % N_NODES
    srcp = jnp.concatenate([src, spread])
    dstp = jnp.concatenate([dst, spread])
    tp = jnp.concatenate([edge_type, jnp.zeros((pad,), jnp.int32)])

    zc = jnp.zeros((CN,), jnp.float32)
    cnt = _k_count(dstp, tp, zc)
    g, scale = _k_scale(srcp, dstp, tp, cnt)
    zacc = jnp.zeros((NODES_P, D), jnp.float32)

    y1, o1 = _tc_layer(emb, None, w1, root1, b1)
    a1 = _k_agg(y1.reshape(N_R * N_NODES, D), g, dstp, scale, zacc)
    y2, o2 = _tc_layer(o1, (a1[0, :N_NODES], a1[1, :N_NODES]), w2, root2, b2)
    a2 = _k_agg(y2.reshape(N_R * N_NODES, D), g, dstp, scale, zacc)
    return _tc_combine(o2, a2[0, :N_NODES], a2[1, :N_NODES])


# speedup vs baseline: 2.3158x; 1.0364x over previous
import functools

import jax
import jax.numpy as jnp
from jax import lax
from jax.experimental import pallas as pl
from jax.experimental.pallas import tpu as pltpu
from jax.experimental.pallas import tpu_sc as plsc

N_NODES = 10000
N_R = 16
D = 128
E = 320000

NC = 2
NS = 16
L = 16
NW = NC * NS

PAD_DST = N_NODES
NODES_P = 10016
RPS_A = 632
RPS_T = NODES_P - 15 * RPS_A
CN = 10112 * N_R
CNS = CN // NS
BLK = 128
NBLK = 81
NSTEP = NBLK // 3
EPW = NBLK * BLK
EP = EPW * NW

_mesh = plsc.VectorSubcoreMesh(core_axis_name="c", subcore_axis_name="s")

_GDN = lax.GatherDimensionNumbers(
    offset_dims=(), collapsed_slice_dims=(0,), start_index_map=(0,))


def _dg(v, idx):
    return lax.gather(v, idx[:, None], _GDN, (1,),
                      mode=lax.GatherScatterMode.PROMISE_IN_BOUNDS)


def _splat(v, i):
    return _dg(v, jnp.full((L,), i, jnp.int32))


@functools.partial(
    pl.kernel,
    out_type=jax.ShapeDtypeStruct((NC * CN,), jnp.float32),
    mesh=_mesh,
    scratch_types=[
        pltpu.VMEM((BLK,), jnp.int32),
        pltpu.VMEM((BLK,), jnp.int32),
        pltpu.VMEM((BLK,), jnp.int32),
        pltpu.VMEM((BLK,), jnp.float32),
        pltpu.VMEM_SHARED((CN,), jnp.float32),
    ],
)
def _k_count(dst_hbm, t_hbm, zc_hbm, cnt_hbm, dst_v, t_v, idx_v, ones_v, cnt_sh):
    cid = lax.axis_index("c")
    sid = lax.axis_index("s")
    wid = sid * NC + cid

    pltpu.sync_copy(zc_hbm.at[pl.ds(sid * CNS, CNS)],
                    cnt_sh.at[pl.ds(sid * CNS, CNS)])
    plsc.subcore_barrier()

    base = wid * EPW

    @pl.loop(0, NBLK)
    def _(blk):
        off = base + blk * BLK
        pltpu.sync_copy(t_hbm.at[pl.ds(off, BLK)], t_v)
        pltpu.sync_copy(dst_hbm.at[pl.ds(off, BLK)], dst_v)
        for q in range(BLK // L):
            sl = pl.ds(q * L, L)
            idx_v[sl] = dst_v[sl] * N_R + t_v[sl]
            pos = lax.iota(jnp.int32, L) + (off + q * L)
            ones_v[sl] = jnp.where(pos < E, 1.0, 0.0)
        pltpu.sync_copy(ones_v, cnt_sh.at[idx_v], add=True)

    plsc.subcore_barrier()
    pltpu.sync_copy(cnt_sh.at[pl.ds(sid * CNS, CNS)],
                    cnt_hbm.at[pl.ds(cid * CN + sid * CNS, CNS)])


@functools.partial(
    pl.kernel,
    out_type=[jax.ShapeDtypeStruct((EP,), jnp.int32),
              jax.ShapeDtypeStruct((EP,), jnp.float32)],
    mesh=_mesh,
    scratch_types=(
        [pltpu.VMEM((BLK,), jnp.int32)] * 9
        + [pltpu.VMEM((BLK,), jnp.int32)] * 9
        + [pltpu.VMEM((BLK,), jnp.float32)] * 9
        + [pltpu.SemaphoreType.DMA] * 9
    ),
)
def _k_scale(src_hbm, dst_hbm, t_hbm, cnt_hbm, g_hbm, sc_hbm,
             sr0, sr1, sr2, dd0, dd1, dd2, tt0, tt1, tt2,
             gg0, gg1, gg2, ia0, ia1, ia2, ib0, ib1, ib2,
             c00, c01, c02, c10, c11, c12, sc0, sc1, sc2,
             si0, si1, si2, sg0, sg1, sg2, sw0, sw1, sw2):
    srv = [sr0, sr1, sr2]
    dv = [dd0, dd1, dd2]
    tv = [tt0, tt1, tt2]
    gv = [gg0, gg1, gg2]
    i0v = [ia0, ia1, ia2]
    i1v = [ib0, ib1, ib2]
    c0v = [c00, c01, c02]
    c1v = [c10, c11, c12]
    scv = [sc0, sc1, sc2]
    sem_i = [si0, si1, si2]
    sem_g = [sg0, sg1, sg2]
    sem_w = [sw0, sw1, sw2]

    cid = lax.axis_index("c")
    sid = lax.axis_index("s")
    wid = sid * NC + cid
    base = wid * EPW

    def idx_start(k, p):
        off = base + k * BLK
        pltpu.async_copy(src_hbm.at[pl.ds(off, BLK)], srv[p], sem_i[p])
        pltpu.async_copy(dst_hbm.at[pl.ds(off, BLK)], dv[p], sem_i[p])
        pltpu.async_copy(t_hbm.at[pl.ds(off, BLK)], tv[p], sem_i[p])

    def stage_b(p):
        pltpu.make_async_copy(src_hbm.at[pl.ds(0, BLK)], srv[p], sem_i[p]).wait()
        pltpu.make_async_copy(dst_hbm.at[pl.ds(0, BLK)], dv[p], sem_i[p]).wait()
        pltpu.make_async_copy(t_hbm.at[pl.ds(0, BLK)], tv[p], sem_i[p]).wait()
        for q in range(BLK // L):
            sl = pl.ds(q * L, L)
            i0 = dv[p][sl] * N_R + tv[p][sl]
            i0v[p][sl] = i0
            i1v[p][sl] = i0 + CN
            gv[p][sl] = srv[p][sl] * (N_R + 1) + tv[p][sl]
        pltpu.async_copy(cnt_hbm.at[i0v[p]], c0v[p], sem_g[p])
        pltpu.async_copy(cnt_hbm.at[i1v[p]], c1v[p], sem_g[p])

    def stage_c(k, p):
        off = base + k * BLK
        pltpu.make_async_copy(cnt_hbm.at[i0v[p]], c0v[p], sem_g[p]).wait()
        pltpu.make_async_copy(cnt_hbm.at[i1v[p]], c1v[p], sem_g[p]).wait()
        for q in range(BLK // L):
            sl = pl.ds(q * L, L)
            pos = lax.iota(jnp.int32, L) + (off + q * L)
            s = 1.0 / jnp.maximum(c0v[p][sl] + c1v[p][sl], 1.0)
            scv[p][sl] = jnp.where(pos < E, s, 0.0)
        pltpu.async_copy(gv[p], g_hbm.at[pl.ds(off, BLK)], sem_w[p])
        pltpu.async_copy(scv[p], sc_hbm.at[pl.ds(off, BLK)], sem_w[p])

    def w_wait(p):
        pltpu.make_async_copy(gv[p], g_hbm.at[pl.ds(0, BLK)], sem_w[p]).wait()
        pltpu.make_async_copy(scv[p], sc_hbm.at[pl.ds(0, BLK)], sem_w[p]).wait()

    idx_start(0, 0)
    idx_start(1, 1)
    stage_b(0)

    @pl.loop(0, NSTEP)
    def _(step):
        for u in range(3):
            k = step * 3 + u
            p, pn, pn2 = u, (u + 1) % 3, (u + 2) % 3

            if u == 0:
                @pl.when(step > 0)
                def _():
                    w_wait(pn2)
            else:
                w_wait(pn2)

            if u == 0:
                idx_start(k + 2, pn2)
            else:
                @pl.when(step < NSTEP - 1)
                def _():
                    idx_start(k + 2, pn2)

            if u < 2:
                stage_b(pn)
            else:
                @pl.when(step < NSTEP - 1)
                def _():
                    stage_b(pn)

            stage_c(k, p)

    w_wait((NBLK - 1) % 3)


@functools.partial(
    pl.kernel,
    out_type=jax.ShapeDtypeStruct((NC, NODES_P, D), jnp.float32),
    mesh=_mesh,
    scratch_types=(
        [pltpu.VMEM((BLK,), jnp.int32)] * 3
        + [pltpu.VMEM((BLK,), jnp.int32)] * 3
        + [pltpu.VMEM((BLK,), jnp.float32)] * 3
        + [pltpu.VMEM((BLK, D), jnp.float32)] * 3
        + [pltpu.VMEM_SHARED((NODES_P, D), jnp.float32)]
        + [pltpu.SemaphoreType.DMA] * 9
    ),
)
def _k_agg(y_hbm, g_hbm, dst_hbm, sc_hbm, z_hbm, acc_hbm,
           g0, g1, g2, d0, d1, d2, s0, s1, s2, r0, r1, r2, acc_sh,
           si0, si1, si2, sr0, sr1, sr2, ss0, ss1, ss2):
    gv = [g0, g1, g2]
    dv = [d0, d1, d2]
    sv = [s0, s1, s2]
    rv = [r0, r1, r2]
    sem_i = [si0, si1, si2]
    sem_r = [sr0, sr1, sr2]
    sem_s = [ss0, ss1, ss2]

    cid = lax.axis_index("c")
    sid = lax.axis_index("s")
    wid = sid * NC + cid
    base = wid * EPW

    def idx_start(k, p):
        off = base + k * BLK
        pltpu.async_copy(g_hbm.at[pl.ds(off, BLK)], gv[p], sem_i[p])
        pltpu.async_copy(dst_hbm.at[pl.ds(off, BLK)], dv[p], sem_i[p])
        pltpu.async_copy(sc_hbm.at[pl.ds(off, BLK)], sv[p], sem_i[p])

    def idx_wait(p):
        pltpu.make_async_copy(g_hbm.at[pl.ds(0, BLK)], gv[p], sem_i[p]).wait()
        pltpu.make_async_copy(dst_hbm.at[pl.ds(0, BLK)], dv[p], sem_i[p]).wait()
        pltpu.make_async_copy(sc_hbm.at[pl.ds(0, BLK)], sv[p], sem_i[p]).wait()

    def gather_start(p):
        pltpu.async_copy(y_hbm.at[gv[p]], rv[p], sem_r[p])

    def gather_wait(p):
        pltpu.make_async_copy(y_hbm.at[gv[p]], rv[p], sem_r[p]).wait()

    def scat_start(p):
        pltpu.async_copy(rv[p], acc_sh.at[dv[p]], sem_s[p], add=True)

    def scat_wait(p):
        pltpu.make_async_copy(rv[p], acc_sh.at[dv[p]], sem_s[p]).wait()

    @pl.when(sid < 15)
    def _():
        pltpu.sync_copy(z_hbm.at[pl.ds(sid * RPS_A, RPS_A)],
                        acc_sh.at[pl.ds(sid * RPS_A, RPS_A)])

    @pl.when(sid == 15)
    def _():
        pltpu.sync_copy(z_hbm.at[pl.ds(15 * RPS_A, RPS_T)],
                        acc_sh.at[pl.ds(15 * RPS_A, RPS_T)])

    plsc.subcore_barrier()

    idx_start(0, 0)
    idx_start(1, 1)
    idx_wait(0)
    gather_start(0)

    @pl.loop(0, NSTEP)
    def _(step):
        for u in range(3):
            k = step * 3 + u
            p, pn, pn2 = u, (u + 1) % 3, (u + 2) % 3

            if u == 0:
                @pl.when(step > 0)
                def _():
                    scat_wait(pn2)
            else:
                scat_wait(pn2)

            if u == 0:
                idx_start(k + 2, pn2)
            else:
                @pl.when(step < NSTEP - 1)
                def _():
                    idx_start(k + 2, pn2)

            if u < 2:
                idx_wait(pn)
                gather_start(pn)
            else:
                @pl.when(step < NSTEP - 1)
                def _():
                    idx_wait(pn)
                    gather_start(pn)

            gather_wait(p)
            for g in range(BLK // L):
                sch = sv[p][pl.ds(g * L, L)]
                for i in range(L):
                    e = g * L + i
                    s = _splat(sch, i)
                    for c in range(D // L):
                        sl = pl.ds(c * L, L)
                        rv[p][e, sl] = rv[p][e, sl] * s
            scat_start(p)

    scat_wait((NBLK - 1) % 3)
    plsc.subcore_barrier()

    @pl.when(sid < 15)
    def _():
        pltpu.sync_copy(acc_sh.at[pl.ds(sid * RPS_A, RPS_A)],
                        acc_hbm.at[cid].at[pl.ds(sid * RPS_A, RPS_A)])

    @pl.when(sid == 15)
    def _():
        pltpu.sync_copy(acc_sh.at[pl.ds(15 * RPS_A, RPS_T)],
                        acc_hbm.at[cid].at[pl.ds(15 * RPS_A, RPS_T)])


BJ = 1000
NBJ = N_NODES // BJ
DC = (N_R + 1) * D


def _tc_matmul1(x, wcat):
    def body(x_ref, w_ref, y_ref):
        y_ref[...] = lax.dot_general(
            x_ref[...].astype(jnp.bfloat16), w_ref[...],
            (((1,), (0,)), ((), ())), preferred_element_type=jnp.float32)

    return pl.pallas_call(
        body,
        grid=(NBJ,),
        in_specs=[pl.BlockSpec((BJ, D), lambda j: (j, 0)),
                  pl.BlockSpec((D, DC), lambda j: (0, 0))],
        out_specs=pl.BlockSpec((BJ, DC), lambda j: (j, 0)),
        out_shape=jax.ShapeDtypeStruct((N_NODES, DC), jnp.float32),
    )(x, wcat)


def _tc_matmul2(y_prev, b_prev, a0, a1, wcat):
    def body(o_ref, b_ref, a0_ref, a1_ref, w_ref, y_ref):
        xb = jnp.maximum(o_ref[...] + b_ref[...] + a0_ref[...] + a1_ref[...],
                         0.0)
        y_ref[...] = lax.dot_general(
            xb.astype(jnp.bfloat16), w_ref[...],
            (((1,), (0,)), ((), ())), preferred_element_type=jnp.float32)

    spec = pl.BlockSpec((BJ, D), lambda j: (j, 0))
    return pl.pallas_call(
        body,
        grid=(NBJ,),
        in_specs=[pl.BlockSpec((BJ, D), lambda j: (j, N_R)),
                  pl.BlockSpec((1, D), lambda j: (0, 0)),
                  spec, spec,
                  pl.BlockSpec((D, DC), lambda j: (0, 0))],
        out_specs=pl.BlockSpec((BJ, DC), lambda j: (j, 0)),
        out_shape=jax.ShapeDtypeStruct((N_NODES, DC), jnp.float32),
    )(y_prev, b_prev.reshape(1, D), a0, a1, wcat)


def _tc_combine(y_prev, b_prev, a0, a1):
    def body(o_ref, b_ref, a0_ref, a1_ref, out_ref):
        out_ref[...] = o_ref[...] + b_ref[...] + a0_ref[...] + a1_ref[...]

    spec = pl.BlockSpec((BJ, D), lambda j: (j, 0))
    return pl.pallas_call(
        body,
        grid=(NBJ,),
        in_specs=[pl.BlockSpec((BJ, D), lambda j: (j, N_R)),
                  pl.BlockSpec((1, D), lambda j: (0, 0)),
                  spec, spec],
        out_specs=spec,
        out_shape=jax.ShapeDtypeStruct((N_NODES, D), jnp.float32),
    )(y_prev, b_prev.reshape(1, D), a0, a1)


def kernel(edge_index, edge_type, emb, w1, root1, b1, w2, root2, b2):
    src = edge_index[0]
    dst = edge_index[1]
    pad = EP - E
    spread = (jnp.arange(pad, dtype=jnp.int32) * 37) % N_NODES
    srcp = jnp.concatenate([src, spread])
    dstp = jnp.concatenate([dst, spread])
    tp = jnp.concatenate([edge_type, jnp.zeros((pad,), jnp.int32)])

    zc = jnp.zeros((CN,), jnp.float32)
    cnt = _k_count(dstp, tp, zc)
    g, scale = _k_scale(srcp, dstp, tp, cnt)
    zacc = jnp.zeros((NODES_P, D), jnp.float32)

    wcat1 = jnp.concatenate(
        [w1.transpose(1, 0, 2).reshape(D, N_R * D), root1],
        axis=1).astype(jnp.bfloat16)
    wcat2 = jnp.concatenate(
        [w2.transpose(1, 0, 2).reshape(D, N_R * D), root2],
        axis=1).astype(jnp.bfloat16)

    y1 = _tc_matmul1(emb, wcat1)
    a1 = _k_agg(y1.reshape(N_NODES * (N_R + 1), D), g, dstp, scale, zacc)
    y2 = _tc_matmul2(y1, b1, a1[0, :N_NODES], a1[1, :N_NODES], wcat2)
    a2 = _k_agg(y2.reshape(N_NODES * (N_R + 1), D), g, dstp, scale, zacc)
    return _tc_combine(y2, b2, a2[0, :N_NODES], a2[1, :N_NODES])


# speedup vs baseline: 2.5099x; 1.0838x over previous
import functools

import jax
import jax.numpy as jnp
from jax import lax
from jax.experimental import pallas as pl
from jax.experimental.pallas import tpu as pltpu
from jax.experimental.pallas import tpu_sc as plsc

N_NODES = 10000
N_R = 16
D = 128
E = 320000

NC = 2
NS = 16
L = 16
NW = NC * NS

PAD_DST = N_NODES
NODES_P = 10016
RPS_A = 632
RPS_T = NODES_P - 15 * RPS_A
CN = 10112 * N_R
CNS = CN // NS
BLK = 128
NBLK = 81
NSTEP = NBLK // 3
EPW = NBLK * BLK
EP = EPW * NW

_mesh = plsc.VectorSubcoreMesh(core_axis_name="c", subcore_axis_name="s")

_GDN = lax.GatherDimensionNumbers(
    offset_dims=(), collapsed_slice_dims=(0,), start_index_map=(0,))


def _dg(v, idx):
    return lax.gather(v, idx[:, None], _GDN, (1,),
                      mode=lax.GatherScatterMode.PROMISE_IN_BOUNDS)


def _splat(v, i):
    return _dg(v, jnp.full((L,), i, jnp.int32))


@functools.partial(
    pl.kernel,
    out_type=jax.ShapeDtypeStruct((NC * CN,), jnp.float32),
    mesh=_mesh,
    scratch_types=(
        [pltpu.VMEM((BLK,), jnp.int32)] * 6
        + [pltpu.VMEM((BLK,), jnp.int32)] * 3
        + [pltpu.VMEM((BLK,), jnp.float32)] * 3
        + [pltpu.VMEM_SHARED((CN,), jnp.float32)]
        + [pltpu.SemaphoreType.DMA] * 6
    ),
)
def _k_count(dst_hbm, t_hbm, zc_hbm, cnt_hbm,
             dd0, dd1, dd2, tt0, tt1, tt2, ix0, ix1, ix2, on0, on1, on2,
             cnt_sh, si0, si1, si2, ss0, ss1, ss2):
    dv = [dd0, dd1, dd2]
    tv = [tt0, tt1, tt2]
    ixv = [ix0, ix1, ix2]
    onv = [on0, on1, on2]
    sem_i = [si0, si1, si2]
    sem_s = [ss0, ss1, ss2]

    cid = lax.axis_index("c")
    sid = lax.axis_index("s")
    wid = sid * NC + cid
    base = wid * EPW

    def idx_start(k, p):
        off = base + k * BLK
        pltpu.async_copy(t_hbm.at[pl.ds(off, BLK)], tv[p], sem_i[p])
        pltpu.async_copy(dst_hbm.at[pl.ds(off, BLK)], dv[p], sem_i[p])

    def idx_wait(p):
        pltpu.make_async_copy(t_hbm.at[pl.ds(0, BLK)], tv[p], sem_i[p]).wait()
        pltpu.make_async_copy(dst_hbm.at[pl.ds(0, BLK)], dv[p], sem_i[p]).wait()

    def scat_wait(p):
        pltpu.make_async_copy(onv[p], cnt_sh.at[ixv[p]], sem_s[p]).wait()

    pltpu.sync_copy(zc_hbm.at[pl.ds(sid * CNS, CNS)],
                    cnt_sh.at[pl.ds(sid * CNS, CNS)])
    plsc.subcore_barrier()

    idx_start(0, 0)
    idx_start(1, 1)

    @pl.loop(0, NSTEP)
    def _(step):
        for u in range(3):
            k = step * 3 + u
            p, pn2 = u, (u + 2) % 3

            if u == 0:
                @pl.when(step > 0)
                def _():
                    scat_wait(pn2)
            else:
                scat_wait(pn2)

            if u == 0:
                idx_start(k + 2, pn2)
            else:
                @pl.when(step < NSTEP - 1)
                def _():
                    idx_start(k + 2, pn2)

            idx_wait(p)
            off = base + k * BLK
            for q in range(BLK // L):
                sl = pl.ds(q * L, L)
                ixv[p][sl] = dv[p][sl] * N_R + tv[p][sl]
                pos = lax.iota(jnp.int32, L) + (off + q * L)
                onv[p][sl] = jnp.where(pos < E, 1.0, 0.0)
            pltpu.async_copy(onv[p], cnt_sh.at[ixv[p]], sem_s[p], add=True)

    scat_wait((NBLK - 1) % 3)
    plsc.subcore_barrier()
    pltpu.sync_copy(cnt_sh.at[pl.ds(sid * CNS, CNS)],
                    cnt_hbm.at[pl.ds(cid * CN + sid * CNS, CNS)])


@functools.partial(
    pl.kernel,
    out_type=[jax.ShapeDtypeStruct((EP,), jnp.int32),
              jax.ShapeDtypeStruct((EP,), jnp.float32)],
    mesh=_mesh,
    scratch_types=(
        [pltpu.VMEM((BLK,), jnp.int32)] * 9
        + [pltpu.VMEM((BLK,), jnp.int32)] * 9
        + [pltpu.VMEM((BLK,), jnp.float32)] * 9
        + [pltpu.SemaphoreType.DMA] * 9
    ),
)
def _k_scale(src_hbm, dst_hbm, t_hbm, cnt_hbm, g_hbm, sc_hbm,
             sr0, sr1, sr2, dd0, dd1, dd2, tt0, tt1, tt2,
             gg0, gg1, gg2, ia0, ia1, ia2, ib0, ib1, ib2,
             c00, c01, c02, c10, c11, c12, sc0, sc1, sc2,
             si0, si1, si2, sg0, sg1, sg2, sw0, sw1, sw2):
    srv = [sr0, sr1, sr2]
    dv = [dd0, dd1, dd2]
    tv = [tt0, tt1, tt2]
    gv = [gg0, gg1, gg2]
    i0v = [ia0, ia1, ia2]
    i1v = [ib0, ib1, ib2]
    c0v = [c00, c01, c02]
    c1v = [c10, c11, c12]
    scv = [sc0, sc1, sc2]
    sem_i = [si0, si1, si2]
    sem_g = [sg0, sg1, sg2]
    sem_w = [sw0, sw1, sw2]

    cid = lax.axis_index("c")
    sid = lax.axis_index("s")
    wid = sid * NC + cid
    base = wid * EPW

    def idx_start(k, p):
        off = base + k * BLK
        pltpu.async_copy(src_hbm.at[pl.ds(off, BLK)], srv[p], sem_i[p])
        pltpu.async_copy(dst_hbm.at[pl.ds(off, BLK)], dv[p], sem_i[p])
        pltpu.async_copy(t_hbm.at[pl.ds(off, BLK)], tv[p], sem_i[p])

    def stage_b(p):
        pltpu.make_async_copy(src_hbm.at[pl.ds(0, BLK)], srv[p], sem_i[p]).wait()
        pltpu.make_async_copy(dst_hbm.at[pl.ds(0, BLK)], dv[p], sem_i[p]).wait()
        pltpu.make_async_copy(t_hbm.at[pl.ds(0, BLK)], tv[p], sem_i[p]).wait()
        for q in range(BLK // L):
            sl = pl.ds(q * L, L)
            i0 = dv[p][sl] * N_R + tv[p][sl]
            i0v[p][sl] = i0
            i1v[p][sl] = i0 + CN
            gv[p][sl] = srv[p][sl] * (N_R + 1) + tv[p][sl]
        pltpu.async_copy(cnt_hbm.at[i0v[p]], c0v[p], sem_g[p])
        pltpu.async_copy(cnt_hbm.at[i1v[p]], c1v[p], sem_g[p])

    def stage_c(k, p):
        off = base + k * BLK
        pltpu.make_async_copy(cnt_hbm.at[i0v[p]], c0v[p], sem_g[p]).wait()
        pltpu.make_async_copy(cnt_hbm.at[i1v[p]], c1v[p], sem_g[p]).wait()
        for q in range(BLK // L):
            sl = pl.ds(q * L, L)
            pos = lax.iota(jnp.int32, L) + (off + q * L)
            s = 1.0 / jnp.maximum(c0v[p][sl] + c1v[p][sl], 1.0)
            scv[p][sl] = jnp.where(pos < E, s, 0.0)
        pltpu.async_copy(gv[p], g_hbm.at[pl.ds(off, BLK)], sem_w[p])
        pltpu.async_copy(scv[p], sc_hbm.at[pl.ds(off, BLK)], sem_w[p])

    def w_wait(p):
        pltpu.make_async_copy(gv[p], g_hbm.at[pl.ds(0, BLK)], sem_w[p]).wait()
        pltpu.make_async_copy(scv[p], sc_hbm.at[pl.ds(0, BLK)], sem_w[p]).wait()

    idx_start(0, 0)
    idx_start(1, 1)
    stage_b(0)

    @pl.loop(0, NSTEP)
    def _(step):
        for u in range(3):
            k = step * 3 + u
            p, pn, pn2 = u, (u + 1) % 3, (u + 2) % 3

            if u == 0:
                @pl.when(step > 0)
                def _():
                    w_wait(pn2)
            else:
                w_wait(pn2)

            if u == 0:
                idx_start(k + 2, pn2)
            else:
                @pl.when(step < NSTEP - 1)
                def _():
                    idx_start(k + 2, pn2)

            if u < 2:
                stage_b(pn)
            else:
                @pl.when(step < NSTEP - 1)
                def _():
                    stage_b(pn)

            stage_c(k, p)

    w_wait((NBLK - 1) % 3)


@functools.partial(
    pl.kernel,
    out_type=jax.ShapeDtypeStruct((NC, NODES_P, D), jnp.float32),
    mesh=_mesh,
    scratch_types=(
        [pltpu.VMEM((BLK,), jnp.int32)] * 3
        + [pltpu.VMEM((BLK,), jnp.int32)] * 3
        + [pltpu.VMEM((BLK,), jnp.float32)] * 3
        + [pltpu.VMEM((BLK, D), jnp.float32)] * 3
        + [pltpu.VMEM_SHARED((NODES_P, D), jnp.float32)]
        + [pltpu.SemaphoreType.DMA] * 9
    ),
)
def _k_agg(y_hbm, g_hbm, dst_hbm, sc_hbm, z_hbm, acc_hbm,
           g0, g1, g2, d0, d1, d2, s0, s1, s2, r0, r1, r2, acc_sh,
           si0, si1, si2, sr0, sr1, sr2, ss0, ss1, ss2):
    gv = [g0, g1, g2]
    dv = [d0, d1, d2]
    sv = [s0, s1, s2]
    rv = [r0, r1, r2]
    sem_i = [si0, si1, si2]
    sem_r = [sr0, sr1, sr2]
    sem_s = [ss0, ss1, ss2]

    cid = lax.axis_index("c")
    sid = lax.axis_index("s")
    wid = sid * NC + cid
    base = wid * EPW

    def idx_start(k, p):
        off = base + k * BLK
        pltpu.async_copy(g_hbm.at[pl.ds(off, BLK)], gv[p], sem_i[p])
        pltpu.async_copy(dst_hbm.at[pl.ds(off, BLK)], dv[p], sem_i[p])
        pltpu.async_copy(sc_hbm.at[pl.ds(off, BLK)], sv[p], sem_i[p])

    def idx_wait(p):
        pltpu.make_async_copy(g_hbm.at[pl.ds(0, BLK)], gv[p], sem_i[p]).wait()
        pltpu.make_async_copy(dst_hbm.at[pl.ds(0, BLK)], dv[p], sem_i[p]).wait()
        pltpu.make_async_copy(sc_hbm.at[pl.ds(0, BLK)], sv[p], sem_i[p]).wait()

    def gather_start(p):
        pltpu.async_copy(y_hbm.at[gv[p]], rv[p], sem_r[p])

    def gather_wait(p):
        pltpu.make_async_copy(y_hbm.at[gv[p]], rv[p], sem_r[p]).wait()

    def scat_start(p):
        pltpu.async_copy(rv[p], acc_sh.at[dv[p]], sem_s[p], add=True)

    def scat_wait(p):
        pltpu.make_async_copy(rv[p], acc_sh.at[dv[p]], sem_s[p]).wait()

    @pl.when(sid < 15)
    def _():
        pltpu.sync_copy(z_hbm.at[pl.ds(sid * RPS_A, RPS_A)],
                        acc_sh.at[pl.ds(sid * RPS_A, RPS_A)])

    @pl.when(sid == 15)
    def _():
        pltpu.sync_copy(z_hbm.at[pl.ds(15 * RPS_A, RPS_T)],
                        acc_sh.at[pl.ds(15 * RPS_A, RPS_T)])

    plsc.subcore_barrier()

    idx_start(0, 0)
    idx_start(1, 1)
    idx_wait(0)
    gather_start(0)

    @pl.loop(0, NSTEP)
    def _(step):
        for u in range(3):
            k = step * 3 + u
            p, pn, pn2 = u, (u + 1) % 3, (u + 2) % 3

            if u == 0:
                @pl.when(step > 0)
                def _():
                    scat_wait(pn2)
            else:
                scat_wait(pn2)

            if u == 0:
                idx_start(k + 2, pn2)
            else:
                @pl.when(step < NSTEP - 1)
                def _():
                    idx_start(k + 2, pn2)

            if u < 2:
                idx_wait(pn)
                gather_start(pn)
            else:
                @pl.when(step < NSTEP - 1)
                def _():
                    idx_wait(pn)
                    gather_start(pn)

            gather_wait(p)
            for g in range(BLK // L):
                sch = sv[p][pl.ds(g * L, L)]
                for i in range(L):
                    e = g * L + i
                    s = _splat(sch, i)
                    for c in range(D // L):
                        sl = pl.ds(c * L, L)
                        rv[p][e, sl] = rv[p][e, sl] * s
            scat_start(p)

    scat_wait((NBLK - 1) % 3)
    plsc.subcore_barrier()

    @pl.when(sid < 15)
    def _():
        pltpu.sync_copy(acc_sh.at[pl.ds(sid * RPS_A, RPS_A)],
                        acc_hbm.at[cid].at[pl.ds(sid * RPS_A, RPS_A)])

    @pl.when(sid == 15)
    def _():
        pltpu.sync_copy(acc_sh.at[pl.ds(15 * RPS_A, RPS_T)],
                        acc_hbm.at[cid].at[pl.ds(15 * RPS_A, RPS_T)])


BJ = 1000
NBJ = N_NODES // BJ
DC = (N_R + 1) * D


def _tc_matmul1(x, wcat):
    def body(x_ref, w_ref, y_ref):
        y_ref[...] = lax.dot_general(
            x_ref[...].astype(jnp.bfloat16), w_ref[...],
            (((1,), (0,)), ((), ())), preferred_element_type=jnp.float32)

    return pl.pallas_call(
        body,
        grid=(NBJ,),
        in_specs=[pl.BlockSpec((BJ, D), lambda j: (j, 0)),
                  pl.BlockSpec((D, DC), lambda j: (0, 0))],
        out_specs=pl.BlockSpec((BJ, DC), lambda j: (j, 0)),
        out_shape=jax.ShapeDtypeStruct((N_NODES, DC), jnp.float32),
    )(x, wcat)


def _tc_matmul2(y_prev, b_prev, a0, a1, wcat):
    def body(o_ref, b_ref, a0_ref, a1_ref, w_ref, y_ref):
        xb = jnp.maximum(o_ref[...] + b_ref[...] + a0_ref[...] + a1_ref[...],
                         0.0)
        y_ref[...] = lax.dot_general(
            xb.astype(jnp.bfloat16), w_ref[...],
            (((1,), (0,)), ((), ())), preferred_element_type=jnp.float32)

    spec = pl.BlockSpec((BJ, D), lambda j: (j, 0))
    return pl.pallas_call(
        body,
        grid=(NBJ,),
        in_specs=[pl.BlockSpec((BJ, D), lambda j: (j, N_R)),
                  pl.BlockSpec((1, D), lambda j: (0, 0)),
                  spec, spec,
                  pl.BlockSpec((D, DC), lambda j: (0, 0))],
        out_specs=pl.BlockSpec((BJ, DC), lambda j: (j, 0)),
        out_shape=jax.ShapeDtypeStruct((N_NODES, DC), jnp.float32),
    )(y_prev, b_prev.reshape(1, D), a0, a1, wcat)


def _tc_combine(y_prev, b_prev, a0, a1):
    def body(o_ref, b_ref, a0_ref, a1_ref, out_ref):
        out_ref[...] = o_ref[...] + b_ref[...] + a0_ref[...] + a1_ref[...]

    spec = pl.BlockSpec((BJ, D), lambda j: (j, 0))
    return pl.pallas_call(
        body,
        grid=(NBJ,),
        in_specs=[pl.BlockSpec((BJ, D), lambda j: (j, N_R)),
                  pl.BlockSpec((1, D), lambda j: (0, 0)),
                  spec, spec],
        out_specs=spec,
        out_shape=jax.ShapeDtypeStruct((N_NODES, D), jnp.float32),
    )(y_prev, b_prev.reshape(1, D), a0, a1)


def kernel(edge_index, edge_type, emb, w1, root1, b1, w2, root2, b2):
    src = edge_index[0]
    dst = edge_index[1]
    pad = EP - E
    spread = (jnp.arange(pad, dtype=jnp.int32) * 37) % N_NODES
    srcp = jnp.concatenate([src, spread])
    dstp = jnp.concatenate([dst, spread])
    tp = jnp.concatenate([edge_type, jnp.zeros((pad,), jnp.int32)])

    zc = jnp.zeros((CN,), jnp.float32)
    cnt = _k_count(dstp, tp, zc)
    g, scale = _k_scale(srcp, dstp, tp, cnt)
    zacc = jnp.zeros((NODES_P, D), jnp.float32)

    wcat1 = jnp.concatenate(
        [w1.transpose(1, 0, 2).reshape(D, N_R * D), root1],
        axis=1).astype(jnp.bfloat16)
    wcat2 = jnp.concatenate(
        [w2.transpose(1, 0, 2).reshape(D, N_R * D), root2],
        axis=1).astype(jnp.bfloat16)

    y1 = _tc_matmul1(emb, wcat1)
    a1 = _k_agg(y1.reshape(N_NODES * (N_R + 1), D), g, dstp, scale, zacc)
    y2 = _tc_matmul2(y1, b1, a1[0, :N_NODES], a1[1, :N_NODES], wcat2)
    a2 = _k_agg(y2.reshape(N_NODES * (N_R + 1), D), g, dstp, scale, zacc)
    return _tc_combine(y2, b2, a2[0, :N_NODES], a2[1, :N_NODES])


# speedup vs baseline: 2.5147x; 1.0019x over previous
import functools

import jax
import jax.numpy as jnp
from jax import lax
from jax.experimental import pallas as pl
from jax.experimental.pallas import tpu as pltpu
from jax.experimental.pallas import tpu_sc as plsc

N_NODES = 10000
N_R = 16
D = 128
E = 320000

NC = 2
NS = 16
L = 16
NW = NC * NS

NODES_P = 10016
RPS_A = 632
RPS_T = NODES_P - 15 * RPS_A
CN = 10112 * N_R
CNS = CN // NS
BLK = 128
NBLK = 81
NSTEP = NBLK // 3
EPW = NBLK * BLK
EP = EPW * NW

_mesh = plsc.VectorSubcoreMesh(core_axis_name="c", subcore_axis_name="s")

_GDN = lax.GatherDimensionNumbers(
    offset_dims=(), collapsed_slice_dims=(0,), start_index_map=(0,))


def _dg(v, idx):
    return lax.gather(v, idx[:, None], _GDN, (1,),
                      mode=lax.GatherScatterMode.PROMISE_IN_BOUNDS)


def _splat(v, i):
    return _dg(v, jnp.full((L,), i, jnp.int32))


@functools.partial(
    pl.kernel,
    out_type=jax.ShapeDtypeStruct((NC * CN,), jnp.float32),
    mesh=_mesh,
    scratch_types=(
        [pltpu.VMEM((BLK,), jnp.int32)] * 6
        + [pltpu.VMEM((BLK,), jnp.int32)] * 3
        + [pltpu.VMEM((BLK,), jnp.float32)] * 3
        + [pltpu.VMEM_SHARED((CN,), jnp.float32)]
        + [pltpu.SemaphoreType.DMA] * 6
    ),
)
def _k_count(dst_hbm, t_hbm, zc_hbm, cnt_hbm,
             dd0, dd1, dd2, tt0, tt1, tt2, ix0, ix1, ix2, on0, on1, on2,
             cnt_sh, si0, si1, si2, ss0, ss1, ss2):
    dv = [dd0, dd1, dd2]
    tv = [tt0, tt1, tt2]
    ixv = [ix0, ix1, ix2]
    onv = [on0, on1, on2]
    sem_i = [si0, si1, si2]
    sem_s = [ss0, ss1, ss2]

    cid = lax.axis_index("c")
    sid = lax.axis_index("s")
    wid = sid * NC + cid
    base = wid * EPW

    def idx_start(k, p):
        off = base + k * BLK
        pltpu.async_copy(t_hbm.at[pl.ds(off, BLK)], tv[p], sem_i[p])
        pltpu.async_copy(dst_hbm.at[pl.ds(off, BLK)], dv[p], sem_i[p])

    def idx_wait(p):
        pltpu.make_async_copy(t_hbm.at[pl.ds(0, BLK)], tv[p], sem_i[p]).wait()
        pltpu.make_async_copy(dst_hbm.at[pl.ds(0, BLK)], dv[p], sem_i[p]).wait()

    def scat_wait(p):
        pltpu.make_async_copy(onv[p], cnt_sh.at[ixv[p]], sem_s[p]).wait()

    pltpu.sync_copy(zc_hbm.at[pl.ds(sid * CNS, CNS)],
                    cnt_sh.at[pl.ds(sid * CNS, CNS)])
    plsc.subcore_barrier()

    idx_start(0, 0)
    idx_start(1, 1)

    @pl.loop(0, NSTEP)
    def _(step):
        for u in range(3):
            k = step * 3 + u
            p, pn2 = u, (u + 2) % 3

            if u == 0:
                @pl.when(step > 0)
                def _():
                    scat_wait(pn2)
            else:
                scat_wait(pn2)

            if u == 0:
                idx_start(k + 2, pn2)
            else:
                @pl.when(step < NSTEP - 1)
                def _():
                    idx_start(k + 2, pn2)

            idx_wait(p)
            off = base + k * BLK
            for q in range(BLK // L):
                sl = pl.ds(q * L, L)
                ixv[p][sl] = dv[p][sl] * N_R + tv[p][sl]
                pos = lax.iota(jnp.int32, L) + (off + q * L)
                onv[p][sl] = jnp.where(pos < E, 1.0, 0.0)
            pltpu.async_copy(onv[p], cnt_sh.at[ixv[p]], sem_s[p], add=True)

    scat_wait((NBLK - 1) % 3)
    plsc.subcore_barrier()
    pltpu.sync_copy(cnt_sh.at[pl.ds(sid * CNS, CNS)],
                    cnt_hbm.at[pl.ds(cid * CN + sid * CNS, CNS)])


@functools.partial(
    pl.kernel,
    out_type=[jax.ShapeDtypeStruct((EP,), jnp.int32),
              jax.ShapeDtypeStruct((EP,), jnp.float32)],
    mesh=_mesh,
    scratch_types=(
        [pltpu.VMEM((BLK,), jnp.int32)] * 9
        + [pltpu.VMEM((BLK,), jnp.int32)] * 9
        + [pltpu.VMEM((BLK,), jnp.float32)] * 9
        + [pltpu.SemaphoreType.DMA] * 9
    ),
)
def _k_scale(src_hbm, dst_hbm, t_hbm, cnt_hbm, g_hbm, sc_hbm,
             sr0, sr1, sr2, dd0, dd1, dd2, tt0, tt1, tt2,
             gg0, gg1, gg2, ia0, ia1, ia2, ib0, ib1, ib2,
             c00, c01, c02, c10, c11, c12, sc0, sc1, sc2,
             si0, si1, si2, sg0, sg1, sg2, sw0, sw1, sw2):
    srv = [sr0, sr1, sr2]
    dv = [dd0, dd1, dd2]
    tv = [tt0, tt1, tt2]
    gv = [gg0, gg1, gg2]
    i0v = [ia0, ia1, ia2]
    i1v = [ib0, ib1, ib2]
    c0v = [c00, c01, c02]
    c1v = [c10, c11, c12]
    scv = [sc0, sc1, sc2]
    sem_i = [si0, si1, si2]
    sem_g = [sg0, sg1, sg2]
    sem_w = [sw0, sw1, sw2]

    cid = lax.axis_index("c")
    sid = lax.axis_index("s")
    wid = sid * NC + cid
    base = wid * EPW

    def idx_start(k, p):
        off = base + k * BLK
        pltpu.async_copy(src_hbm.at[pl.ds(off, BLK)], srv[p], sem_i[p])
        pltpu.async_copy(dst_hbm.at[pl.ds(off, BLK)], dv[p], sem_i[p])
        pltpu.async_copy(t_hbm.at[pl.ds(off, BLK)], tv[p], sem_i[p])

    def stage_b(p):
        pltpu.make_async_copy(src_hbm.at[pl.ds(0, BLK)], srv[p], sem_i[p]).wait()
        pltpu.make_async_copy(dst_hbm.at[pl.ds(0, BLK)], dv[p], sem_i[p]).wait()
        pltpu.make_async_copy(t_hbm.at[pl.ds(0, BLK)], tv[p], sem_i[p]).wait()
        for q in range(BLK // L):
            sl = pl.ds(q * L, L)
            i0 = dv[p][sl] * N_R + tv[p][sl]
            i0v[p][sl] = i0
            i1v[p][sl] = i0 + CN
            gv[p][sl] = srv[p][sl] * (N_R + 1) + tv[p][sl]
        pltpu.async_copy(cnt_hbm.at[i0v[p]], c0v[p], sem_g[p])
        pltpu.async_copy(cnt_hbm.at[i1v[p]], c1v[p], sem_g[p])

    def stage_c(k, p):
        off = base + k * BLK
        pltpu.make_async_copy(cnt_hbm.at[i0v[p]], c0v[p], sem_g[p]).wait()
        pltpu.make_async_copy(cnt_hbm.at[i1v[p]], c1v[p], sem_g[p]).wait()
        for q in range(BLK // L):
            sl = pl.ds(q * L, L)
            pos = lax.iota(jnp.int32, L) + (off + q * L)
            s = 1.0 / jnp.maximum(c0v[p][sl] + c1v[p][sl], 1.0)
            scv[p][sl] = jnp.where(pos < E, s, 0.0)
        pltpu.async_copy(gv[p], g_hbm.at[pl.ds(off, BLK)], sem_w[p])
        pltpu.async_copy(scv[p], sc_hbm.at[pl.ds(off, BLK)], sem_w[p])

    def w_wait(p):
        pltpu.make_async_copy(gv[p], g_hbm.at[pl.ds(0, BLK)], sem_w[p]).wait()
        pltpu.make_async_copy(scv[p], sc_hbm.at[pl.ds(0, BLK)], sem_w[p]).wait()

    idx_start(0, 0)
    idx_start(1, 1)
    stage_b(0)

    @pl.loop(0, NSTEP)
    def _(step):
        for u in range(3):
            k = step * 3 + u
            p, pn, pn2 = u, (u + 1) % 3, (u + 2) % 3

            if u == 0:
                @pl.when(step > 0)
                def _():
                    w_wait(pn2)
            else:
                w_wait(pn2)

            if u == 0:
                idx_start(k + 2, pn2)
            else:
                @pl.when(step < NSTEP - 1)
                def _():
                    idx_start(k + 2, pn2)

            if u < 2:
                stage_b(pn)
            else:
                @pl.when(step < NSTEP - 1)
                def _():
                    stage_b(pn)

            stage_c(k, p)

    w_wait((NBLK - 1) % 3)


@functools.partial(
    pl.kernel,
    out_type=jax.ShapeDtypeStruct((NC, NODES_P, D), jnp.float32),
    mesh=_mesh,
    scratch_types=(
        [pltpu.VMEM((BLK,), jnp.int32)] * 3
        + [pltpu.VMEM((BLK,), jnp.int32)] * 3
        + [pltpu.VMEM((BLK,), jnp.float32)] * 3
        + [pltpu.VMEM((BLK, D), jnp.float32)] * 3
        + [pltpu.VMEM_SHARED((NODES_P, D), jnp.float32)]
        + [pltpu.SemaphoreType.DMA] * 9
    ),
)
def _k_agg(y_hbm, g_hbm, dst_hbm, sc_hbm, z_hbm, acc_hbm,
           g0, g1, g2, d0, d1, d2, s0, s1, s2, r0, r1, r2, acc_sh,
           si0, si1, si2, sr0, sr1, sr2, ss0, ss1, ss2):
    gv = [g0, g1, g2]
    dv = [d0, d1, d2]
    sv = [s0, s1, s2]
    rv = [r0, r1, r2]
    sem_i = [si0, si1, si2]
    sem_r = [sr0, sr1, sr2]
    sem_s = [ss0, ss1, ss2]

    cid = lax.axis_index("c")
    sid = lax.axis_index("s")
    wid = sid * NC + cid
    base = wid * EPW

    def idx_start(k, p):
        off = base + k * BLK
        pltpu.async_copy(g_hbm.at[pl.ds(off, BLK)], gv[p], sem_i[p])
        pltpu.async_copy(dst_hbm.at[pl.ds(off, BLK)], dv[p], sem_i[p])
        pltpu.async_copy(sc_hbm.at[pl.ds(off, BLK)], sv[p], sem_i[p])

    def idx_wait(p):
        pltpu.make_async_copy(g_hbm.at[pl.ds(0, BLK)], gv[p], sem_i[p]).wait()
        pltpu.make_async_copy(dst_hbm.at[pl.ds(0, BLK)], dv[p], sem_i[p]).wait()
        pltpu.make_async_copy(sc_hbm.at[pl.ds(0, BLK)], sv[p], sem_i[p]).wait()

    def gather_start(p):
        pltpu.async_copy(y_hbm.at[gv[p]], rv[p], sem_r[p])

    def gather_wait(p):
        pltpu.make_async_copy(y_hbm.at[gv[p]], rv[p], sem_r[p]).wait()

    def scat_start(p):
        pltpu.async_copy(rv[p], acc_sh.at[dv[p]], sem_s[p], add=True)

    def scat_wait(p):
        pltpu.make_async_copy(rv[p], acc_sh.at[dv[p]], sem_s[p]).wait()

    @pl.when(sid < 15)
    def _():
        pltpu.sync_copy(z_hbm.at[pl.ds(sid * RPS_A, RPS_A)],
                        acc_sh.at[pl.ds(sid * RPS_A, RPS_A)])

    @pl.when(sid == 15)
    def _():
        pltpu.sync_copy(z_hbm.at[pl.ds(15 * RPS_A, RPS_T)],
                        acc_sh.at[pl.ds(15 * RPS_A, RPS_T)])

    plsc.subcore_barrier()

    idx_start(0, 0)
    idx_start(1, 1)
    idx_wait(0)
    gather_start(0)

    @pl.loop(0, NSTEP)
    def _(step):
        for u in range(3):
            k = step * 3 + u
            p, pn, pn2 = u, (u + 1) % 3, (u + 2) % 3

            if u == 0:
                @pl.when(step > 0)
                def _():
                    scat_wait(pn2)
            else:
                scat_wait(pn2)

            if u == 0:
                idx_start(k + 2, pn2)
            else:
                @pl.when(step < NSTEP - 1)
                def _():
                    idx_start(k + 2, pn2)

            if u < 2:
                idx_wait(pn)
                gather_start(pn)
            else:
                @pl.when(step < NSTEP - 1)
                def _():
                    idx_wait(pn)
                    gather_start(pn)

            gather_wait(p)
            for g in range(BLK // L):
                sch = sv[p][pl.ds(g * L, L)]
                for i in range(L):
                    e = g * L + i
                    s = _splat(sch, i)
                    for c in range(D // L):
                        sl = pl.ds(c * L, L)
                        rv[p][e, sl] = rv[p][e, sl] * s
            scat_start(p)

    scat_wait((NBLK - 1) % 3)
    plsc.subcore_barrier()

    @pl.when(sid < 15)
    def _():
        pltpu.sync_copy(acc_sh.at[pl.ds(sid * RPS_A, RPS_A)],
                        acc_hbm.at[cid].at[pl.ds(sid * RPS_A, RPS_A)])

    @pl.when(sid == 15)
    def _():
        pltpu.sync_copy(acc_sh.at[pl.ds(15 * RPS_A, RPS_T)],
                        acc_hbm.at[cid].at[pl.ds(15 * RPS_A, RPS_T)])


BJ = 1000
NBJ = N_NODES // BJ
DC = (N_R + 1) * D


def _tc_matmul1(x, wcat):
    def body(x_ref, w_ref, y_ref):
        y_ref[...] = lax.dot_general(
            x_ref[...].astype(jnp.bfloat16), w_ref[...],
            (((1,), (0,)), ((), ())), preferred_element_type=jnp.float32)

    return pl.pallas_call(
        body,
        grid=(NBJ,),
        in_specs=[pl.BlockSpec((BJ, D), lambda j: (j, 0)),
                  pl.BlockSpec((D, DC), lambda j: (0, 0))],
        out_specs=pl.BlockSpec((BJ, DC), lambda j: (j, 0)),
        out_shape=jax.ShapeDtypeStruct((N_NODES, DC), jnp.float32),
    )(x, wcat)


def _tc_matmul2(y_prev, b_prev, a0, a1, wcat):
    def body(o_ref, b_ref, a0_ref, a1_ref, w_ref, y_ref):
        xb = jnp.maximum(o_ref[...] + b_ref[...] + a0_ref[...] + a1_ref[...],
                         0.0)
        y_ref[...] = lax.dot_general(
            xb.astype(jnp.bfloat16), w_ref[...],
            (((1,), (0,)), ((), ())), preferred_element_type=jnp.float32)

    spec = pl.BlockSpec((BJ, D), lambda j: (j, 0))
    return pl.pallas_call(
        body,
        grid=(NBJ,),
        in_specs=[pl.BlockSpec((BJ, D), lambda j: (j, N_R)),
                  pl.BlockSpec((1, D), lambda j: (0, 0)),
                  spec, spec,
                  pl.BlockSpec((D, DC), lambda j: (0, 0))],
        out_specs=pl.BlockSpec((BJ, DC), lambda j: (j, 0)),
        out_shape=jax.ShapeDtypeStruct((N_NODES, DC), jnp.float32),
    )(y_prev, b_prev.reshape(1, D), a0, a1, wcat)


def _tc_combine(y_prev, b_prev, a0, a1):
    def body(o_ref, b_ref, a0_ref, a1_ref, out_ref):
        out_ref[...] = o_ref[...] + b_ref[...] + a0_ref[...] + a1_ref[...]

    spec = pl.BlockSpec((BJ, D), lambda j: (j, 0))
    return pl.pallas_call(
        body,
        grid=(NBJ,),
        in_specs=[pl.BlockSpec((BJ, D), lambda j: (j, N_R)),
                  pl.BlockSpec((1, D), lambda j: (0, 0)),
                  spec, spec],
        out_specs=spec,
        out_shape=jax.ShapeDtypeStruct((N_NODES, D), jnp.float32),
    )(y_prev, b_prev.reshape(1, D), a0, a1)


def kernel(edge_index, edge_type, emb, w1, root1, b1, w2, root2, b2):
    src = edge_index[0]
    dst = edge_index[1]
    pad = EP - E
    spread = (jnp.arange(pad, dtype=jnp.int32) * 37) % N_NODES
    srcp = jnp.concatenate([src, spread])
    dstp = jnp.concatenate([dst, spread])
    tp = jnp.concatenate([edge_type, jnp.zeros((pad,), jnp.int32)])

    zc = jnp.zeros((CN,), jnp.float32)
    cnt = _k_count(dstp, tp, zc)
    g, scale = _k_scale(srcp, dstp, tp, cnt)
    zacc = jnp.zeros((NODES_P, D), jnp.float32)

    wcat1 = jnp.concatenate(
        [w1.transpose(1, 0, 2).reshape(D, N_R * D), root1],
        axis=1).astype(jnp.bfloat16)
    wcat2 = jnp.concatenate(
        [w2.transpose(1, 0, 2).reshape(D, N_R * D), root2],
        axis=1).astype(jnp.bfloat16)

    y1 = _tc_matmul1(emb, wcat1)
    a1 = _k_agg(y1.reshape(N_NODES * (N_R + 1), D), g, dstp, scale, zacc)
    y2 = _tc_matmul2(y1, b1, a1[0, :N_NODES], a1[1, :N_NODES], wcat2)
    a2 = _k_agg(y2.reshape(N_NODES * (N_R + 1), D), g, dstp, scale, zacc)
    return _tc_combine(y2, b2, a2[0, :N_NODES], a2[1, :N_NODES])
